# Initial kernel scaffold; baseline (speedup 1.0000x reference)
import jax
import jax.numpy as jnp
from jax.experimental import pallas as pl


def kernel(x, edge_index, edge_attr, W1, We1, as1, ad1, ae1, b1, W2, We2, as2, ad2, ae2, b2):
    raise NotImplementedError("write your pallas kernel here")


# speedup vs baseline: 18.5389x; 18.5389x over previous
import functools
import jax
import jax.numpy as jnp
from jax import lax
from jax.experimental import pallas as pl
from jax.experimental.pallas import tpu as pltpu
from jax.experimental.pallas import tpu_sc as plsc

N = 10000
E = 320000
IN = 128
H = 4
HID = 256
ED = 16
C1 = 64
C2 = 256

NC, NS, L = 2, 16, 16
NPAD = 10240
EPT = E // NS
BE = 400
G = 80
RW = 144
ROWS_PT = NPAD // NS
F32 = jnp.float32
I32 = jnp.int32

_mesh = plsc.VectorSubcoreMesh(core_axis_name="c", subcore_axis_name="s",
                               num_cores=NC, num_subcores=NS)
_sc_params = pltpu.CompilerParams(needs_layout_passes=False,
                                  use_tc_tiling_on_sc=False)


def _mm(a, b, block_m):
    M, K = a.shape
    Nn = b.shape[1]

    def body(a_ref, b_ref, o_ref):
        o_ref[...] = jnp.dot(a_ref[...], b_ref[...],
                             preferred_element_type=F32)

    return pl.pallas_call(
        body,
        grid=(M // block_m,),
        in_specs=[pl.BlockSpec((block_m, K), lambda m: (m, 0)),
                  pl.BlockSpec((K, Nn), lambda m: (0, 0))],
        out_specs=pl.BlockSpec((block_m, Nn), lambda m: (m, 0)),
        out_shape=jax.ShapeDtypeStruct((M, Nn), F32),
    )(a, b)


def _zero_vmem(ref, nrows, width):
    zv = jnp.zeros((L,), F32)

    def zb(r, c):
        for j in range(width // L):
            ref[r, pl.ds(j * L, L)] = zv
        return c

    lax.fori_loop(0, nrows, zb, 0)


def _zero_shared(zbuf, sh, base):
    def zb(i, c):
        pltpu.sync_copy(zbuf, sh.at[pl.ds(base + i * 16, 16)])
        return c

    lax.fori_loop(0, ROWS_PT // 16, zb, 0)


def _splat(vec_ref, r):
    return plsc.load_gather(vec_ref, [jnp.full((L,), r, I32)])


def _edge_weight(asg_v, adg_v, ae_v, rid16, eidx16, h, ae_col):
    a_s = plsc.load_gather(asg_v, [rid16, jnp.full((L,), h, I32)])
    a_d = plsc.load_gather(adg_v, [rid16, jnp.full((L,), 4 + h, I32)])
    a_e = plsc.load_gather(ae_v, [eidx16, jnp.full((L,), ae_col, I32)])
    pre = a_s + a_d + a_e
    return jnp.exp(jnp.where(pre >= 0, pre, F32(0.2) * pre))


def _sc1_body(src_h, dst_h, ae_h, astab_h, ea_h, xp1a_h, xp1b_h,
              big_out, esum_out,
              srcs_v, dsts_v, ae_v, asg_v, adg_v, ea_g, srci_v, dsti_v,
              w0_v, w1_v, rows_v, zero_v, zesum_v, acc_sh, esum_sh,
              sem, sem2, sem3):
    cid = lax.axis_index("c")
    sid = lax.axis_index("s")
    base = sid * ROWS_PT
    lane = lax.iota(I32, L)

    _zero_vmem(zero_v, 16, RW)
    _zero_vmem(zesum_v, 16, 16)
    _zero_shared(zero_v, acc_sh, base)

    @pl.when(cid == 0)
    def _():
        _zero_shared(zesum_v, esum_sh, base)

    plsc.subcore_barrier()

    def run(xp_tbl, ha, hb, do_esum):
        def stage_body(s, c0):
            estart = sid * EPT + s * BE
            pltpu.sync_copy(src_h.at[pl.ds(estart, BE)], srcs_v)
            pltpu.sync_copy(dst_h.at[pl.ds(estart, BE)], dsts_v)
            pltpu.sync_copy(ae_h.at[pl.ds(estart, BE)], ae_v)

            def group_body(g, c1):
                for k in range(G // L):
                    i0 = g * G + k * L
                    srci_v[pl.ds(k * L, L)] = srcs_v[pl.ds(i0, L)]
                    dsti_v[pl.ds(k * L, L)] = dsts_v[pl.ds(i0, L)]

                d1 = pltpu.async_copy(xp_tbl.at[srci_v], rows_v, sem)
                d2 = pltpu.async_copy(astab_h.at[srci_v], asg_v, sem2)
                d3 = pltpu.async_copy(astab_h.at[dsti_v], adg_v, sem3)
                d2.wait()
                d3.wait()

                for k in range(G // L):
                    i0 = g * G + k * L
                    rid16 = lane + k * L
                    eidx16 = lane + i0
                    w0_v[pl.ds(k * L, L)] = _edge_weight(
                        asg_v, adg_v, ae_v, rid16, eidx16, ha, ha)
                    w1_v[pl.ds(k * L, L)] = _edge_weight(
                        asg_v, adg_v, ae_v, rid16, eidx16, hb, hb)

                d1.wait()

                def mul_body(r, c2):
                    wva = _splat(w0_v, r)
                    wvb = _splat(w1_v, r)
                    one = jnp.ones((L,), F32)
                    zero = jnp.zeros((L,), F32)
                    wmix = jnp.where(
                        lane == 0, wva,
                        jnp.where(lane == 1, wvb,
                                  jnp.where(lane == 2, one, zero)))
                    for j in range(RW // L):
                        wj = wva if j < 4 else (wvb if j < 8 else wmix)
                        rows_v[r, pl.ds(j * L, L)] = (
                            rows_v[r, pl.ds(j * L, L)] * wj)
                    return c2

                lax.fori_loop(0, G, mul_body, 0)
                pltpu.sync_copy(rows_v, acc_sh.at[dsti_v], add=True)
                if do_esum:
                    estart2 = sid * EPT + s * BE + g * G
                    pltpu.sync_copy(ea_h.at[pl.ds(estart2, G)], ea_g)
                    pltpu.sync_copy(ea_g, esum_sh.at[dsti_v], add=True)
                return c1

            lax.fori_loop(0, BE // G, group_body, 0)
            return c0

        lax.fori_loop(0, EPT // BE, stage_body, 0)

    @pl.when(cid == 0)
    def _():
        run(xp1a_h, 0, 1, True)

    @pl.when(cid == 1)
    def _():
        run(xp1b_h, 2, 3, False)

    plsc.subcore_barrier()

    def dump(c):
        def db(i, cc):
            rows = pl.ds(base + i * 32, 32)
            pltpu.sync_copy(acc_sh.at[rows], big_out.at[c, rows])
            return cc

        lax.fori_loop(0, ROWS_PT // 32, db, 0)

    @pl.when(cid == 0)
    def _():
        dump(0)

        def db2(i, cc):
            rows = pl.ds(base + i * 32, 32)
            pltpu.sync_copy(esum_sh.at[rows], esum_out.at[rows])
            return cc

        lax.fori_loop(0, ROWS_PT // 32, db2, 0)

    @pl.when(cid == 1)
    def _():
        dump(1)


def _sc_layer1(src, dst, ae12, astab1, edge_attr, xp1a, xp1b):
    fn = pl.kernel(
        _sc1_body,
        out_type=(jax.ShapeDtypeStruct((NC, NPAD, RW), F32),
                  jax.ShapeDtypeStruct((NPAD, 16), F32)),
        mesh=_mesh,
        compiler_params=_sc_params,
        scratch_types=[
            pltpu.VMEM((BE,), I32),
            pltpu.VMEM((BE,), I32),
            pltpu.VMEM((BE, 8), F32),
            pltpu.VMEM((G, 16), F32),
            pltpu.VMEM((G, 16), F32),
            pltpu.VMEM((G, 16), F32),
            pltpu.VMEM((G,), I32),
            pltpu.VMEM((G,), I32),
            pltpu.VMEM((G,), F32),
            pltpu.VMEM((G,), F32),
            pltpu.VMEM((G, RW), F32),
            pltpu.VMEM((16, RW), F32),
            pltpu.VMEM((16, 16), F32),
            pltpu.VMEM_SHARED((NPAD, RW), F32),
            pltpu.VMEM_SHARED((NPAD, 16), F32),
            pltpu.SemaphoreType.DMA,
            pltpu.SemaphoreType.DMA,
            pltpu.SemaphoreType.DMA,
        ],
    )
    return fn(src, dst, ae12, astab1, edge_attr, xp1a, xp1b)


def _sc2_body(src_h, dst_h, ae_h, astab_h,
              xq0, xq1, xq2, xq3, xq4, xq5, xq6, xq7,
              big_out,
              srcs_v, dsts_v, ae_v, asg_v, adg_v, srci_v, dsti_v,
              w0_v, rows_v, zero_v, acc_sh, sem, sem2, sem3):
    cid = lax.axis_index("c")
    sid = lax.axis_index("s")
    base = sid * ROWS_PT
    lane = lax.iota(I32, L)
    xq = (xq0, xq1, xq2, xq3, xq4, xq5, xq6, xq7)

    _zero_vmem(zero_v, 16, RW)

    def run(xp_tbl, h):
        def stage_body(s, c0):
            estart = sid * EPT + s * BE
            pltpu.sync_copy(src_h.at[pl.ds(estart, BE)], srcs_v)
            pltpu.sync_copy(dst_h.at[pl.ds(estart, BE)], dsts_v)
            pltpu.sync_copy(ae_h.at[pl.ds(estart, BE)], ae_v)

            def group_body(g, c1):
                for k in range(G // L):
                    i0 = g * G + k * L
                    srci_v[pl.ds(k * L, L)] = srcs_v[pl.ds(i0, L)]
                    dsti_v[pl.ds(k * L, L)] = dsts_v[pl.ds(i0, L)]

                d1 = pltpu.async_copy(xp_tbl.at[srci_v], rows_v, sem)
                d2 = pltpu.async_copy(astab_h.at[srci_v], asg_v, sem2)
                d3 = pltpu.async_copy(astab_h.at[dsti_v], adg_v, sem3)
                d2.wait()
                d3.wait()

                for k in range(G // L):
                    i0 = g * G + k * L
                    rid16 = lane + k * L
                    eidx16 = lane + i0
                    w0_v[pl.ds(k * L, L)] = _edge_weight(
                        asg_v, adg_v, ae_v, rid16, eidx16, h, 4 + h)

                d1.wait()

                def mul_body(r, c2):
                    wva = _splat(w0_v, r)
                    zero = jnp.zeros((L,), F32)
                    wmix = jnp.where(lane == 0, wva, zero)
                    for j in range(RW // L):
                        wj = wva if j < 8 else wmix
                        rows_v[r, pl.ds(j * L, L)] = (
                            rows_v[r, pl.ds(j * L, L)] * wj)
                    return c2

                lax.fori_loop(0, G, mul_body, 0)
                pltpu.sync_copy(rows_v, acc_sh.at[dsti_v], add=True)
                return c1

            lax.fori_loop(0, BE // G, group_body, 0)
            return c0

        lax.fori_loop(0, EPT // BE, stage_body, 0)

    for h in range(4):
        _zero_shared(zero_v, acc_sh, base)
        plsc.subcore_barrier()

        @pl.when(cid == 0)
        def _(h=h):
            run(xq[2 * h + 0], h)

        @pl.when(cid == 1)
        def _(h=h):
            run(xq[2 * h + 1], h)

        plsc.subcore_barrier()

        @pl.when(cid == 0)
        def _(h=h):
            def db(i, cc):
                rows = pl.ds(base + i * 32, 32)
                pltpu.sync_copy(acc_sh.at[rows], big_out.at[h, 0, rows])
                return cc

            lax.fori_loop(0, ROWS_PT // 32, db, 0)

        @pl.when(cid == 1)
        def _(h=h):
            def db(i, cc):
                rows = pl.ds(base + i * 32, 32)
                pltpu.sync_copy(acc_sh.at[rows], big_out.at[h, 1, rows])
                return cc

            lax.fori_loop(0, ROWS_PT // 32, db, 0)


def _sc_layer2(src, dst, ae12, astab2, xqs):
    fn = pl.kernel(
        _sc2_body,
        out_type=jax.ShapeDtypeStruct((H, NC, NPAD, RW), F32),
        mesh=_mesh,
        compiler_params=_sc_params,
        scratch_types=[
            pltpu.VMEM((BE,), I32),
            pltpu.VMEM((BE,), I32),
            pltpu.VMEM((BE, 8), F32),
            pltpu.VMEM((G, 16), F32),
            pltpu.VMEM((G, 16), F32),
            pltpu.VMEM((G,), I32),
            pltpu.VMEM((G,), I32),
            pltpu.VMEM((G,), F32),
            pltpu.VMEM((G, RW), F32),
            pltpu.VMEM((16, RW), F32),
            pltpu.VMEM_SHARED((NPAD, RW), F32),
            pltpu.SemaphoreType.DMA,
            pltpu.SemaphoreType.DMA,
            pltpu.SemaphoreType.DMA,
        ],
    )
    return fn(src, dst, ae12, astab2, *xqs)


def _mid(big1a, big1b, esum, cat1, vcat, w2cat, b1):
    NB = 1000

    def body(b1a_ref, b1b_ref, es_ref, cat_ref, v_ref, w_ref, bias_ref,
             xp2t_ref, asad2_ref):
        cnt = jnp.maximum(b1a_ref[:, 130:131], 1.0)
        sumae = jnp.dot(es_ref[...], v_ref[...],
                        preferred_element_type=F32)
        a_s = cat_ref[:, 256:260]
        a_d = cat_ref[:, 260:264]
        pre = a_s + a_d + sumae[:, 0:4] / cnt
        wl = jnp.exp(jnp.where(pre >= 0, pre, 0.2 * pre))
        hs = []
        for h in range(4):
            xp_h = cat_ref[:, h * 64:(h + 1) * 64]
            bigref = b1a_ref if h < 2 else b1b_ref
            bg = bigref[:, (h % 2) * 64:((h % 2) + 1) * 64]
            den = bigref[:, 128 + (h % 2):129 + (h % 2)]
            wlh = wl[:, h:h + 1]
            hs.append((bg + wlh * xp_h) / (den + wlh + 1e-16))
        hcat = jnp.concatenate(hs, axis=1) + bias_ref[...]
        hrelu = jnp.maximum(hcat, 0.0)
        res = jnp.dot(hrelu, w_ref[...], preferred_element_type=F32)
        ones = jnp.ones((NB, 1), F32)
        zpad = jnp.zeros((NB, 15), F32)
        for t in range(8):
            xp2t_ref[t] = jnp.concatenate(
                [res[:, t * 128:(t + 1) * 128], ones, zpad], axis=1)
        asad2_ref[...] = jnp.concatenate(
            [res[:, 1024:1032], cnt, jnp.zeros((NB, 7), F32)], axis=1)

    return pl.pallas_call(
        body,
        grid=(N // NB,),
        in_specs=[
            pl.BlockSpec((NB, RW), lambda m: (m, 0)),
            pl.BlockSpec((NB, RW), lambda m: (m, 0)),
            pl.BlockSpec((NB, 16), lambda m: (m, 0)),
            pl.BlockSpec((NB, 272), lambda m: (m, 0)),
            pl.BlockSpec((ED, 8), lambda m: (0, 0)),
            pl.BlockSpec((HID, 1032), lambda m: (0, 0)),
            pl.BlockSpec((1, HID), lambda m: (0, 0)),
        ],
        out_specs=[
            pl.BlockSpec((8, NB, RW), lambda m: (0, m, 0)),
            pl.BlockSpec((NB, 16), lambda m: (m, 0)),
        ],
        out_shape=[
            jax.ShapeDtypeStruct((8, N, RW), F32),
            jax.ShapeDtypeStruct((N, 16), F32),
        ],
    )(big1a, big1b, esum, cat1, vcat, w2cat, b1)


def _post(big2, esum, vcat, asad2, xp2t, b2):
    NB = 1000

    def body(big_ref, es_ref, v_ref, asd_ref, xp_ref, bias_ref, o_ref):
        cnt = jnp.maximum(asd_ref[:, 8:9], 1.0)
        sumae = jnp.dot(es_ref[...], v_ref[...],
                        preferred_element_type=F32)
        a_s = asd_ref[:, 0:4]
        a_d = asd_ref[:, 4:8]
        pre = a_s + a_d + sumae[:, 4:8] / cnt
        wl = jnp.exp(jnp.where(pre >= 0, pre, 0.2 * pre))
        for c in range(2):
            acc = None
            for h in range(4):
                wlh = wl[:, h:h + 1]
                den = big_ref[h, 0][:, 128:129]
                o_hc = ((big_ref[h, c][:, 0:128] +
                         wlh * xp_ref[2 * h + c][:, 0:128]) /
                        (den + wlh + 1e-16))
                acc = o_hc if acc is None else acc + o_hc
            o_ref[:, c * 128:(c + 1) * 128] = (
                0.25 * acc + bias_ref[:, c * 128:(c + 1) * 128])

    return pl.pallas_call(
        body,
        grid=(N // NB,),
        in_specs=[
            pl.BlockSpec((H, NC, NB, RW), lambda m: (0, 0, m, 0)),
            pl.BlockSpec((NB, 16), lambda m: (m, 0)),
            pl.BlockSpec((ED, 8), lambda m: (0, 0)),
            pl.BlockSpec((NB, 16), lambda m: (m, 0)),
            pl.BlockSpec((8, NB, RW), lambda m: (0, m, 0)),
            pl.BlockSpec((1, HID), lambda m: (0, 0)),
        ],
        out_specs=pl.BlockSpec((NB, HID), lambda m: (m, 0)),
        out_shape=jax.ShapeDtypeStruct((N, HID), F32),
    )(big2, esum, vcat, asad2, xp2t, b2)


def kernel(x, edge_index, edge_attr, W1, We1, as1, ad1, ae1, b1,
           W2, We2, as2, ad2, ae2, b2):
    src = edge_index[0]
    dst = edge_index[1]

    Ve1 = (We1.reshape(ED, H, C1) * ae1).sum(-1)
    Ve2 = (We2.reshape(ED, H, C2) * ae2).sum(-1)
    vcat = jnp.concatenate([Ve1, Ve2], axis=1)
    vs1 = (W1.reshape(IN, H, C1) * as1).sum(-1)
    vd1 = (W1.reshape(IN, H, C1) * ad1).sum(-1)
    wcat1 = jnp.concatenate(
        [W1, vs1, vd1, jnp.zeros((IN, 8), F32)], axis=1)
    vs2 = (W2.reshape(HID, H, C2) * as2).sum(-1)
    vd2 = (W2.reshape(HID, H, C2) * ad2).sum(-1)
    w2cat = jnp.concatenate([W2, vs2, vd2], axis=1)

    cat1 = _mm(x, wcat1, 1000)
    ae12 = _mm(edge_attr, vcat, 8000)

    ones3 = jnp.ones((N, 3), F32)
    zpad13 = jnp.zeros((N, 13), F32)
    xp1a = jnp.concatenate([cat1[:, :128], ones3, zpad13], axis=1)
    xp1b = jnp.concatenate([cat1[:, 128:256], ones3, zpad13], axis=1)
    astab1 = jnp.concatenate(
        [cat1[:, 256:264], jnp.zeros((N, 8), F32)], axis=1)

    big1, esum = _sc_layer1(src, dst, ae12, astab1, edge_attr, xp1a, xp1b)
    esum_n = esum[:N]

    xp2t, asad2 = _mid(big1[0, :N], big1[1, :N], esum_n, cat1, vcat,
                       w2cat, b1.reshape(1, HID))

    xqs = [xp2t[t] for t in range(8)]
    big2 = _sc_layer2(src, dst, ae12, asad2, xqs)

    return _post(big2[:, :, :N], esum_n, vcat, asad2, xp2t,
                 b2.reshape(1, HID))


# speedup vs baseline: 25.0279x; 1.3500x over previous
import functools
import jax
import jax.numpy as jnp
from jax import lax
from jax.experimental import pallas as pl
from jax.experimental.pallas import tpu as pltpu
from jax.experimental.pallas import tpu_sc as plsc

N = 10000
E = 320000
IN = 128
H = 4
HID = 256
ED = 16
C1 = 64
C2 = 256

NC, NS, L = 2, 16, 16
NPAD = 10240
EPT = E // NS
EPW = E // (NC * NS)
BE = 400
G = 80
NGR = BE // G
RW = 144
ROWS_PT = NPAD // NS
F32 = jnp.float32
I32 = jnp.int32

_mesh = plsc.VectorSubcoreMesh(core_axis_name="c", subcore_axis_name="s",
                               num_cores=NC, num_subcores=NS)
_sc_params = pltpu.CompilerParams(needs_layout_passes=False,
                                  use_tc_tiling_on_sc=False)


def _mm(a, b, block_m):
    M, K = a.shape
    Nn = b.shape[1]

    def body(a_ref, b_ref, o_ref):
        o_ref[...] = jnp.dot(a_ref[...], b_ref[...],
                             preferred_element_type=F32)

    return pl.pallas_call(
        body,
        grid=(M // block_m,),
        in_specs=[pl.BlockSpec((block_m, K), lambda m: (m, 0)),
                  pl.BlockSpec((K, Nn), lambda m: (0, 0))],
        out_specs=pl.BlockSpec((block_m, Nn), lambda m: (m, 0)),
        out_shape=jax.ShapeDtypeStruct((M, Nn), F32),
    )(a, b)


def _zero_vmem(ref, nrows, width):
    zv = jnp.zeros((L,), F32)

    def zb(r, c):
        for j in range(width // L):
            ref[r, pl.ds(j * L, L)] = zv
        return c

    lax.fori_loop(0, nrows, zb, 0)


def _zero_shared(zbuf, sh, base):
    def zb(i, c):
        pltpu.sync_copy(zbuf, sh.at[pl.ds(base + i * 16, 16)])
        return c

    lax.fori_loop(0, ROWS_PT // 16, zb, 0)


def _splat(vec_ref, r):
    return plsc.load_gather(vec_ref, [jnp.full((L,), r, I32)])


def _edge_weight(asg_v, adg_v, ae_v, rid16, eidx16, h, ae_col):
    a_s = plsc.load_gather(asg_v, [rid16, jnp.full((L,), h, I32)])
    a_d = plsc.load_gather(adg_v, [rid16, jnp.full((L,), 4 + h, I32)])
    a_e = plsc.load_gather(ae_v, [eidx16, jnp.full((L,), ae_col, I32)])
    pre = a_s + a_d + a_e
    return jnp.exp(jnp.where(pre >= 0, pre, F32(0.2) * pre))


def _stage_pipeline(l1, lane, srcs_v, dsts_v, ae_v, bufs, acc_sh,
                    xp_h, astab_h, off16, ha, hb, aec0, aec1):
    desc = {}

    def fire(gi):
        b = bufs[gi % 2]
        for k in range(G // L):
            i0 = gi * G + k * L
            s16 = srcs_v[pl.ds(i0, L)]
            b["srci"][pl.ds(k * L, L)] = s16
            b["srco"][pl.ds(k * L, L)] = s16 + off16
            b["dsti"][pl.ds(k * L, L)] = dsts_v[pl.ds(i0, L)]
        desc[(gi, "r")] = pltpu.async_copy(
            xp_h.at[b["srco"]], b["rows"], b["semr"])
        desc[(gi, "a")] = pltpu.async_copy(
            astab_h.at[b["srci"]], b["asg"], b["sema"])
        desc[(gi, "d")] = pltpu.async_copy(
            astab_h.at[b["dsti"]], b["adg"], b["semd"])

    def process(gi):
        b = bufs[gi % 2]
        desc[(gi, "a")].wait()
        desc[(gi, "d")].wait()
        for k in range(G // L):
            i0 = gi * G + k * L
            rid16 = lane + k * L
            eidx16 = lane + i0
            b["w0"][pl.ds(k * L, L)] = _edge_weight(
                b["asg"], b["adg"], ae_v, rid16, eidx16, ha, aec0)
            if l1:
                b["w1"][pl.ds(k * L, L)] = _edge_weight(
                    b["asg"], b["adg"], ae_v, rid16, eidx16, hb, aec1)
        desc[(gi, "r")].wait()
        rows_v = b["rows"]
        w0_v = b["w0"]
        w1_v = b["w1"] if l1 else None

        def mul_body(r, c2):
            wva = _splat(w0_v, r)
            one = jnp.ones((L,), F32)
            zero = jnp.zeros((L,), F32)
            if l1:
                wvb = _splat(w1_v, r)
                wmix = jnp.where(
                    lane == 0, wva,
                    jnp.where(lane == 1, wvb,
                              jnp.where(lane == 2, one, zero)))
            else:
                wvb = wva
                wmix = jnp.where(lane == 0, wva, zero)
            for j in range(RW // L):
                wj = wva if j < 4 else (wvb if j < 8 else wmix)
                rows_v[r, pl.ds(j * L, L)] = (
                    rows_v[r, pl.ds(j * L, L)] * wj)
            return c2

        lax.fori_loop(0, G, mul_body, 0)
        desc[(gi, "s")] = pltpu.async_copy(
            b["rows"], acc_sh.at[b["dsti"]], b["semw"], add=True)

    def drain(gi):
        desc[(gi, "s")].wait()

    fire(0)
    fire(1)
    process(0)
    for gi in range(2, NGR):
        drain(gi - 2)
        fire(gi)
        process(gi - 1)
    process(NGR - 1)
    drain(NGR - 2)
    drain(NGR - 1)


def _mk_bufs(args):
    names = ("srci", "srco", "dsti", "w0", "w1", "asg", "adg", "rows",
             "semr", "sema", "semd", "semw")
    a, b = {}, {}
    it = iter(args)
    for n in names:
        a[n] = next(it)
        b[n] = next(it)
    return (a, b)


_BUF_TYPES = [
    pltpu.VMEM((G,), I32), pltpu.VMEM((G,), I32),
    pltpu.VMEM((G,), I32), pltpu.VMEM((G,), I32),
    pltpu.VMEM((G,), I32), pltpu.VMEM((G,), I32),
    pltpu.VMEM((G,), F32), pltpu.VMEM((G,), F32),
    pltpu.VMEM((G,), F32), pltpu.VMEM((G,), F32),
    pltpu.VMEM((G, 16), F32), pltpu.VMEM((G, 16), F32),
    pltpu.VMEM((G, 16), F32), pltpu.VMEM((G, 16), F32),
    pltpu.VMEM((G, RW), F32), pltpu.VMEM((G, RW), F32),
    pltpu.SemaphoreType.DMA, pltpu.SemaphoreType.DMA,
    pltpu.SemaphoreType.DMA, pltpu.SemaphoreType.DMA,
    pltpu.SemaphoreType.DMA, pltpu.SemaphoreType.DMA,
    pltpu.SemaphoreType.DMA, pltpu.SemaphoreType.DMA,
]


def _sc1_body(src_h, dst_h, ae_h, astab_h, xp_h, big_out,
              srcs_v, dsts_v, ae_v, zero_v, acc_sh, *bufargs):
    cid = lax.axis_index("c")
    sid = lax.axis_index("s")
    base = sid * ROWS_PT
    lane = lax.iota(I32, L)
    bufs = _mk_bufs(bufargs)

    _zero_vmem(zero_v, 16, RW)
    _zero_shared(zero_v, acc_sh, base)
    plsc.subcore_barrier()

    ha = cid * 2
    hb = ha + 1
    off16 = jnp.full((L,), cid * N, I32)

    def stage_body(s, c0):
        estart = sid * EPT + s * BE
        pltpu.sync_copy(src_h.at[pl.ds(estart, BE)], srcs_v)
        pltpu.sync_copy(dst_h.at[pl.ds(estart, BE)], dsts_v)
        pltpu.sync_copy(ae_h.at[pl.ds(estart, BE)], ae_v)
        _stage_pipeline(True, lane, srcs_v, dsts_v, ae_v, bufs, acc_sh,
                        xp_h, astab_h, off16, ha, hb, ha, hb)
        return c0

    lax.fori_loop(0, EPT // BE, stage_body, 0)
    plsc.subcore_barrier()

    def db(i, cc):
        rows = pl.ds(base + i * 32, 32)
        orow = pl.ds(cid * NPAD + base + i * 32, 32)
        pltpu.sync_copy(acc_sh.at[rows], big_out.at[orow])
        return cc

    lax.fori_loop(0, ROWS_PT // 32, db, 0)


def _sc_layer1(src, dst, ae12, astab1, xp1cat):
    fn = pl.kernel(
        _sc1_body,
        out_type=jax.ShapeDtypeStruct((NC * NPAD, RW), F32),
        mesh=_mesh,
        compiler_params=_sc_params,
        scratch_types=[
            pltpu.VMEM((BE,), I32),
            pltpu.VMEM((BE,), I32),
            pltpu.VMEM((BE, 8), F32),
            pltpu.VMEM((16, RW), F32),
            pltpu.VMEM_SHARED((NPAD, RW), F32),
        ] + _BUF_TYPES,
    )
    return fn(src, dst, ae12, astab1, xp1cat)


def _sc2_body(src_h, dst_h, ae_h, astab_h, xp_h, big_out,
              srcs_v, dsts_v, ae_v, zero_v, acc_sh, *bufargs):
    cid = lax.axis_index("c")
    sid = lax.axis_index("s")
    base = sid * ROWS_PT
    lane = lax.iota(I32, L)
    bufs = _mk_bufs(bufargs)

    _zero_vmem(zero_v, 16, RW)

    def head_body(h, ch):
        _zero_shared(zero_v, acc_sh, base)
        plsc.subcore_barrier()
        off16 = jnp.full((L,), 1, I32) * ((2 * h + cid) * N)

        def stage_body(s, c0):
            estart = sid * EPT + s * BE
            pltpu.sync_copy(src_h.at[pl.ds(estart, BE)], srcs_v)
            pltpu.sync_copy(dst_h.at[pl.ds(estart, BE)], dsts_v)
            pltpu.sync_copy(ae_h.at[pl.ds(estart, BE)], ae_v)
            _stage_pipeline(False, lane, srcs_v, dsts_v, ae_v, bufs,
                            acc_sh, xp_h, astab_h, off16, h, h,
                            4 + h, 4 + h)
            return c0

        lax.fori_loop(0, EPT // BE, stage_body, 0)
        plsc.subcore_barrier()

        def db(i, cc):
            rows = pl.ds(base + i * 32, 32)
            orow = pl.ds((2 * h + cid) * NPAD + base + i * 32, 32)
            pltpu.sync_copy(acc_sh.at[rows], big_out.at[orow])
            return cc

        lax.fori_loop(0, ROWS_PT // 32, db, 0)
        return ch

    lax.fori_loop(0, H, head_body, 0)


def _sc_layer2(src, dst, ae12, astab2, xp2f):
    fn = pl.kernel(
        _sc2_body,
        out_type=jax.ShapeDtypeStruct((H * NC * NPAD, RW), F32),
        mesh=_mesh,
        compiler_params=_sc_params,
        scratch_types=[
            pltpu.VMEM((BE,), I32),
            pltpu.VMEM((BE,), I32),
            pltpu.VMEM((BE, 8), F32),
            pltpu.VMEM((16, RW), F32),
            pltpu.VMEM_SHARED((NPAD, RW), F32),
        ] + _BUF_TYPES,
    )
    return fn(src, dst, ae12, astab2, xp2f)


def _esum_body(dst_h, ea_h, esum_out, dsts_v, dsti_v, ea_g, zesum_v,
               esum_sh, sem):
    cid = lax.axis_index("c")
    sid = lax.axis_index("s")
    wid = cid * NS + sid
    base = sid * ROWS_PT

    _zero_vmem(zesum_v, 16, 16)
    _zero_shared(zesum_v, esum_sh, base)
    plsc.subcore_barrier()

    def stage_body(s, c0):
        estart = wid * EPW + s * BE
        pltpu.sync_copy(dst_h.at[pl.ds(estart, BE)], dsts_v)

        def group_body(g, c1):
            for k in range(G // L):
                i0 = g * G + k * L
                dsti_v[pl.ds(k * L, L)] = dsts_v[pl.ds(i0, L)]
            pltpu.async_copy(
                ea_h.at[pl.ds(estart + g * G, G)], ea_g, sem).wait()
            pltpu.sync_copy(ea_g, esum_sh.at[dsti_v], add=True)
            return c1

        lax.fori_loop(0, NGR, group_body, 0)
        return c0

    lax.fori_loop(0, EPW // BE, stage_body, 0)
    plsc.subcore_barrier()

    def db(i, cc):
        rows = pl.ds(base + i * 32, 32)
        orow = pl.ds(cid * NPAD + base + i * 32, 32)
        pltpu.sync_copy(esum_sh.at[rows], esum_out.at[orow])
        return cc

    lax.fori_loop(0, ROWS_PT // 32, db, 0)


def _sc_esum(dst, edge_attr):
    fn = pl.kernel(
        _esum_body,
        out_type=jax.ShapeDtypeStruct((NC * NPAD, 16), F32),
        mesh=_mesh,
        compiler_params=_sc_params,
        scratch_types=[
            pltpu.VMEM((BE,), I32),
            pltpu.VMEM((G,), I32),
            pltpu.VMEM((G, 16), F32),
            pltpu.VMEM((16, 16), F32),
            pltpu.VMEM_SHARED((NPAD, 16), F32),
            pltpu.SemaphoreType.DMA,
        ],
    )
    return fn(dst, edge_attr)


def _mid(big1r, esumr, cat1, vcat, w2cat, b1):
    NB = 1000

    def body(big_ref, es_ref, cat_ref, v_ref, w_ref, bias_ref,
             xp2t_ref, asad2_ref):
        cnt = jnp.maximum(big_ref[0][:, 130:131], 1.0)
        es = es_ref[0] + es_ref[1]
        sumae = jnp.dot(es, v_ref[...],
                        preferred_element_type=F32)
        a_s = cat_ref[:, 256:260]
        a_d = cat_ref[:, 260:264]
        pre = a_s + a_d + sumae[:, 0:4] / cnt
        wl = jnp.exp(jnp.where(pre >= 0, pre, 0.2 * pre))
        hs = []
        for h in range(4):
            xp_h = cat_ref[:, h * 64:(h + 1) * 64]
            bigv = big_ref[h // 2]
            bg = bigv[:, (h % 2) * 64:((h % 2) + 1) * 64]
            den = bigv[:, 128 + (h % 2):129 + (h % 2)]
            wlh = wl[:, h:h + 1]
            hs.append((bg + wlh * xp_h) / (den + wlh + 1e-16))
        hcat = jnp.concatenate(hs, axis=1) + bias_ref[...]
        hrelu = jnp.maximum(hcat, 0.0)
        res = jnp.dot(hrelu, w_ref[...], preferred_element_type=F32)
        ones = jnp.ones((NB, 1), F32)
        zpad = jnp.zeros((NB, 15), F32)
        for t in range(8):
            xp2t_ref[t] = jnp.concatenate(
                [res[:, t * 128:(t + 1) * 128], ones, zpad], axis=1)
        asad2_ref[...] = jnp.concatenate(
            [res[:, 1024:1032], cnt, jnp.zeros((NB, 7), F32)], axis=1)

    return pl.pallas_call(
        body,
        grid=(N // NB,),
        in_specs=[
            pl.BlockSpec((NC, NB, RW), lambda m: (0, m, 0)),
            pl.BlockSpec((NC, NB, 16), lambda m: (0, m, 0)),
            pl.BlockSpec((NB, 272), lambda m: (m, 0)),
            pl.BlockSpec((ED, 8), lambda m: (0, 0)),
            pl.BlockSpec((HID, 1032), lambda m: (0, 0)),
            pl.BlockSpec((1, HID), lambda m: (0, 0)),
        ],
        out_specs=[
            pl.BlockSpec((8, NB, RW), lambda m: (0, m, 0)),
            pl.BlockSpec((NB, 16), lambda m: (m, 0)),
        ],
        out_shape=[
            jax.ShapeDtypeStruct((8, N, RW), F32),
            jax.ShapeDtypeStruct((N, 16), F32),
        ],
    )(big1r, esumr, cat1, vcat, w2cat, b1)


def _post(big2r, esumr, vcat, asad2, xp2t, b2):
    NB = 1000

    def body(big_ref, es_ref, v_ref, asd_ref, xp_ref, bias_ref, o_ref):
        cnt = jnp.maximum(asd_ref[:, 8:9], 1.0)
        es = es_ref[0] + es_ref[1]
        sumae = jnp.dot(es, v_ref[...], preferred_element_type=F32)
        a_s = asd_ref[:, 0:4]
        a_d = asd_ref[:, 4:8]
        pre = a_s + a_d + sumae[:, 4:8] / cnt
        wl = jnp.exp(jnp.where(pre >= 0, pre, 0.2 * pre))
        for c in range(2):
            acc = None
            for h in range(4):
                wlh = wl[:, h:h + 1]
                den = big_ref[2 * h][:, 128:129]
                o_hc = ((big_ref[2 * h + c][:, 0:128] +
                         wlh * xp_ref[2 * h + c][:, 0:128]) /
                        (den + wlh + 1e-16))
                acc = o_hc if acc is None else acc + o_hc
            o_ref[:, c * 128:(c + 1) * 128] = (
                0.25 * acc + bias_ref[:, c * 128:(c + 1) * 128])

    return pl.pallas_call(
        body,
        grid=(N // NB,),
        in_specs=[
            pl.BlockSpec((8, NB, RW), lambda m: (0, m, 0)),
            pl.BlockSpec((NC, NB, 16), lambda m: (0, m, 0)),
            pl.BlockSpec((ED, 8), lambda m: (0, 0)),
            pl.BlockSpec((NB, 16), lambda m: (m, 0)),
            pl.BlockSpec((8, NB, RW), lambda m: (0, m, 0)),
            pl.BlockSpec((1, HID), lambda m: (0, 0)),
        ],
        out_specs=pl.BlockSpec((NB, HID), lambda m: (m, 0)),
        out_shape=jax.ShapeDtypeStruct((N, HID), F32),
    )(big2r, esumr, vcat, asad2, xp2t, b2)


def kernel(x, edge_index, edge_attr, W1, We1, as1, ad1, ae1, b1,
           W2, We2, as2, ad2, ae2, b2):
    src = edge_index[0]
    dst = edge_index[1]

    Ve1 = (We1.reshape(ED, H, C1) * ae1).sum(-1)
    Ve2 = (We2.reshape(ED, H, C2) * ae2).sum(-1)
    vcat = jnp.concatenate([Ve1, Ve2], axis=1)
    vs1 = (W1.reshape(IN, H, C1) * as1).sum(-1)
    vd1 = (W1.reshape(IN, H, C1) * ad1).sum(-1)
    wcat1 = jnp.concatenate(
        [W1, vs1, vd1, jnp.zeros((IN, 8), F32)], axis=1)
    vs2 = (W2.reshape(HID, H, C2) * as2).sum(-1)
    vd2 = (W2.reshape(HID, H, C2) * ad2).sum(-1)
    w2cat = jnp.concatenate([W2, vs2, vd2], axis=1)

    cat1 = _mm(x, wcat1, 1000)
    ae12 = _mm(edge_attr, vcat, 8000)

    ones3 = jnp.ones((N, 3), F32)
    zpad13 = jnp.zeros((N, 13), F32)
    xp1cat = jnp.concatenate([
        jnp.concatenate([cat1[:, :128], ones3, zpad13], axis=1),
        jnp.concatenate([cat1[:, 128:256], ones3, zpad13], axis=1),
    ], axis=0)
    astab1 = jnp.concatenate(
        [cat1[:, 256:264], jnp.zeros((N, 8), F32)], axis=1)

    esum = _sc_esum(dst, edge_attr)
    esumr = esum.reshape(NC, NPAD, 16)[:, :N]
    big1f = _sc_layer1(src, dst, ae12, astab1, xp1cat)
    big1r = big1f.reshape(NC, NPAD, RW)[:, :N]

    xp2t, asad2 = _mid(big1r, esumr, cat1, vcat, w2cat,
                       b1.reshape(1, HID))

    big2f = _sc_layer2(src, dst, ae12, asad2, xp2t.reshape(8 * N, RW))
    big2r = big2f.reshape(H * NC, NPAD, RW)[:, :N]

    return _post(big2r, esumr, vcat, asad2, xp2t, b2.reshape(1, HID))


# speedup vs baseline: 27.3056x; 1.0910x over previous
import functools
import jax
import jax.numpy as jnp
from jax import lax
from jax.experimental import pallas as pl
from jax.experimental.pallas import tpu as pltpu
from jax.experimental.pallas import tpu_sc as plsc

N = 10000
E = 320000
IN = 128
H = 4
HID = 256
ED = 16
C1 = 64
C2 = 256

NC, NS, L = 2, 16, 16
NPAD = 10240
EPT = E // NS
EPW = E // (NC * NS)
BE = 400
G = 80
NGR = BE // G
RW = 144
ROWS_PT = NPAD // NS
F32 = jnp.float32
I32 = jnp.int32

_mesh = plsc.VectorSubcoreMesh(core_axis_name="c", subcore_axis_name="s",
                               num_cores=NC, num_subcores=NS)
_sc_params = pltpu.CompilerParams(needs_layout_passes=False,
                                  use_tc_tiling_on_sc=False)


def _mm(a, b, block_m):
    M, K = a.shape
    Nn = b.shape[1]

    def body(a_ref, b_ref, o_ref):
        o_ref[...] = jnp.dot(a_ref[...], b_ref[...],
                             preferred_element_type=F32)

    return pl.pallas_call(
        body,
        grid=(M // block_m,),
        in_specs=[pl.BlockSpec((block_m, K), lambda m: (m, 0)),
                  pl.BlockSpec((K, Nn), lambda m: (0, 0))],
        out_specs=pl.BlockSpec((block_m, Nn), lambda m: (m, 0)),
        out_shape=jax.ShapeDtypeStruct((M, Nn), F32),
    )(a, b)


def _zero_vmem(ref, nrows, width):
    zv = jnp.zeros((L,), F32)

    def zb(r, c):
        for j in range(width // L):
            ref[r, pl.ds(j * L, L)] = zv
        return c

    lax.fori_loop(0, nrows, zb, 0)


def _zero_shared(zbuf, sh, base):
    def zb(i, c):
        pltpu.sync_copy(zbuf, sh.at[pl.ds(base + i * 16, 16)])
        return c

    lax.fori_loop(0, ROWS_PT // 16, zb, 0)


def _splat(vec_ref, r):
    return plsc.load_gather(vec_ref, [jnp.full((L,), r, I32)])


def _edge_weight(asg_v, adg_v, ae_v, rid16, eidx16, h, ae_col):
    a_s = plsc.load_gather(asg_v, [rid16, jnp.full((L,), h, I32)])
    a_d = plsc.load_gather(adg_v, [rid16, jnp.full((L,), 4 + h, I32)])
    a_e = plsc.load_gather(ae_v, [eidx16, jnp.full((L,), ae_col, I32)])
    pre = a_s + a_d + a_e
    return jnp.exp(jnp.where(pre >= 0, pre, F32(0.2) * pre))


def _stage_pipeline(l1, lane, srcs_v, dsts_v, ae_v, bufs, acc_sh,
                    xp_h, astab_h, off16, ha, hb, aec0, aec1):
    desc = {}

    def fire(gi):
        b = bufs[gi % 2]
        for k in range(G // L):
            i0 = gi * G + k * L
            s16 = srcs_v[pl.ds(i0, L)]
            b["srci"][pl.ds(k * L, L)] = s16
            b["srco"][pl.ds(k * L, L)] = s16 + off16
            b["dsti"][pl.ds(k * L, L)] = dsts_v[pl.ds(i0, L)]
        desc[(gi, "r")] = pltpu.async_copy(
            xp_h.at[b["srco"]], b["rows"], b["semr"])
        desc[(gi, "a")] = pltpu.async_copy(
            astab_h.at[b["srci"]], b["asg"], b["sema"])
        desc[(gi, "d")] = pltpu.async_copy(
            astab_h.at[b["dsti"]], b["adg"], b["semd"])

    def process(gi):
        b = bufs[gi % 2]
        desc[(gi, "a")].wait()
        desc[(gi, "d")].wait()
        for k in range(G // L):
            i0 = gi * G + k * L
            rid16 = lane + k * L
            eidx16 = lane + i0
            b["w0"][pl.ds(k * L, L)] = _edge_weight(
                b["asg"], b["adg"], ae_v, rid16, eidx16, ha, aec0)
            if l1:
                b["w1"][pl.ds(k * L, L)] = _edge_weight(
                    b["asg"], b["adg"], ae_v, rid16, eidx16, hb, aec1)
        desc[(gi, "r")].wait()
        rows_v = b["rows"]
        w0_v = b["w0"]
        w1_v = b["w1"] if l1 else None

        @plsc.parallel_loop(0, G, 1, unroll=4)
        def _(r):
            wva = _splat(w0_v, r)
            one = jnp.ones((L,), F32)
            zero = jnp.zeros((L,), F32)
            if l1:
                wvb = _splat(w1_v, r)
                wmix = jnp.where(
                    lane == 0, wva,
                    jnp.where(lane == 1, wvb,
                              jnp.where(lane == 2, one, zero)))
            else:
                wvb = wva
                wmix = jnp.where(lane == 0, wva, zero)
            for j in range(RW // L):
                wj = wva if j < 4 else (wvb if j < 8 else wmix)
                rows_v[r, pl.ds(j * L, L)] = (
                    rows_v[r, pl.ds(j * L, L)] * wj)
        desc[(gi, "s")] = pltpu.async_copy(
            b["rows"], acc_sh.at[b["dsti"]], b["semw"], add=True)

    def drain(gi):
        desc[(gi, "s")].wait()

    fire(0)
    fire(1)
    process(0)
    for gi in range(2, NGR):
        drain(gi - 2)
        fire(gi)
        process(gi - 1)
    process(NGR - 1)
    drain(NGR - 2)
    drain(NGR - 1)


def _mk_bufs(args):
    names = ("srci", "srco", "dsti", "w0", "w1", "asg", "adg", "rows",
             "semr", "sema", "semd", "semw")
    a, b = {}, {}
    it = iter(args)
    for n in names:
        a[n] = next(it)
        b[n] = next(it)
    return (a, b)


_BUF_TYPES = [
    pltpu.VMEM((G,), I32), pltpu.VMEM((G,), I32),
    pltpu.VMEM((G,), I32), pltpu.VMEM((G,), I32),
    pltpu.VMEM((G,), I32), pltpu.VMEM((G,), I32),
    pltpu.VMEM((G,), F32), pltpu.VMEM((G,), F32),
    pltpu.VMEM((G,), F32), pltpu.VMEM((G,), F32),
    pltpu.VMEM((G, 16), F32), pltpu.VMEM((G, 16), F32),
    pltpu.VMEM((G, 16), F32), pltpu.VMEM((G, 16), F32),
    pltpu.VMEM((G, RW), F32), pltpu.VMEM((G, RW), F32),
    pltpu.SemaphoreType.DMA, pltpu.SemaphoreType.DMA,
    pltpu.SemaphoreType.DMA, pltpu.SemaphoreType.DMA,
    pltpu.SemaphoreType.DMA, pltpu.SemaphoreType.DMA,
    pltpu.SemaphoreType.DMA, pltpu.SemaphoreType.DMA,
]


def _sc1_body(src_h, dst_h, ae_h, astab_h, xp_h, big_out,
              srcs_v, dsts_v, ae_v, zero_v, acc_sh, *bufargs):
    cid = lax.axis_index("c")
    sid = lax.axis_index("s")
    base = sid * ROWS_PT
    lane = lax.iota(I32, L)
    bufs = _mk_bufs(bufargs)

    _zero_vmem(zero_v, 16, RW)
    _zero_shared(zero_v, acc_sh, base)
    plsc.subcore_barrier()

    ha = cid * 2
    hb = ha + 1
    off16 = jnp.full((L,), cid * N, I32)

    def stage_body(s, c0):
        estart = sid * EPT + s * BE
        pltpu.sync_copy(src_h.at[pl.ds(estart, BE)], srcs_v)
        pltpu.sync_copy(dst_h.at[pl.ds(estart, BE)], dsts_v)
        pltpu.sync_copy(ae_h.at[pl.ds(estart, BE)], ae_v)
        _stage_pipeline(True, lane, srcs_v, dsts_v, ae_v, bufs, acc_sh,
                        xp_h, astab_h, off16, ha, hb, ha, hb)
        return c0

    lax.fori_loop(0, EPT // BE, stage_body, 0)
    plsc.subcore_barrier()

    def db(i, cc):
        rows = pl.ds(base + i * 32, 32)
        orow = pl.ds(cid * NPAD + base + i * 32, 32)
        pltpu.sync_copy(acc_sh.at[rows], big_out.at[orow])
        return cc

    lax.fori_loop(0, ROWS_PT // 32, db, 0)


def _sc_layer1(src, dst, ae12, astab1, xp1cat):
    fn = pl.kernel(
        _sc1_body,
        out_type=jax.ShapeDtypeStruct((NC * NPAD, RW), F32),
        mesh=_mesh,
        compiler_params=_sc_params,
        scratch_types=[
            pltpu.VMEM((BE,), I32),
            pltpu.VMEM((BE,), I32),
            pltpu.VMEM((BE, 8), F32),
            pltpu.VMEM((16, RW), F32),
            pltpu.VMEM_SHARED((NPAD, RW), F32),
        ] + _BUF_TYPES,
    )
    return fn(src, dst, ae12, astab1, xp1cat)


def _sc2_body(src_h, dst_h, ae_h, astab_h, xp_h, big_out,
              srcs_v, dsts_v, ae_v, zero_v, acc_sh, *bufargs):
    cid = lax.axis_index("c")
    sid = lax.axis_index("s")
    base = sid * ROWS_PT
    lane = lax.iota(I32, L)
    bufs = _mk_bufs(bufargs)

    _zero_vmem(zero_v, 16, RW)

    def head_body(h, ch):
        _zero_shared(zero_v, acc_sh, base)
        plsc.subcore_barrier()
        off16 = jnp.full((L,), 1, I32) * ((2 * h + cid) * N)

        def stage_body(s, c0):
            estart = sid * EPT + s * BE
            pltpu.sync_copy(src_h.at[pl.ds(estart, BE)], srcs_v)
            pltpu.sync_copy(dst_h.at[pl.ds(estart, BE)], dsts_v)
            pltpu.sync_copy(ae_h.at[pl.ds(estart, BE)], ae_v)
            _stage_pipeline(False, lane, srcs_v, dsts_v, ae_v, bufs,
                            acc_sh, xp_h, astab_h, off16, h, h,
                            4 + h, 4 + h)
            return c0

        lax.fori_loop(0, EPT // BE, stage_body, 0)
        plsc.subcore_barrier()

        def db(i, cc):
            rows = pl.ds(base + i * 32, 32)
            orow = pl.ds((2 * h + cid) * NPAD + base + i * 32, 32)
            pltpu.sync_copy(acc_sh.at[rows], big_out.at[orow])
            return cc

        lax.fori_loop(0, ROWS_PT // 32, db, 0)
        return ch

    lax.fori_loop(0, H, head_body, 0)


def _sc_layer2(src, dst, ae12, astab2, xp2f):
    fn = pl.kernel(
        _sc2_body,
        out_type=jax.ShapeDtypeStruct((H * NC * NPAD, RW), F32),
        mesh=_mesh,
        compiler_params=_sc_params,
        scratch_types=[
            pltpu.VMEM((BE,), I32),
            pltpu.VMEM((BE,), I32),
            pltpu.VMEM((BE, 8), F32),
            pltpu.VMEM((16, RW), F32),
            pltpu.VMEM_SHARED((NPAD, RW), F32),
        ] + _BUF_TYPES,
    )
    return fn(src, dst, ae12, astab2, xp2f)


def _esum_body(dst_h, ea_h, esum_out, dsts_v, dsti_v, ea_g, zesum_v,
               esum_sh, sem):
    cid = lax.axis_index("c")
    sid = lax.axis_index("s")
    wid = cid * NS + sid
    base = sid * ROWS_PT

    _zero_vmem(zesum_v, 16, 16)
    _zero_shared(zesum_v, esum_sh, base)
    plsc.subcore_barrier()

    def stage_body(s, c0):
        estart = wid * EPW + s * BE
        pltpu.sync_copy(dst_h.at[pl.ds(estart, BE)], dsts_v)

        def group_body(g, c1):
            for k in range(G // L):
                i0 = g * G + k * L
                dsti_v[pl.ds(k * L, L)] = dsts_v[pl.ds(i0, L)]
            pltpu.async_copy(
                ea_h.at[pl.ds(estart + g * G, G)], ea_g, sem).wait()
            pltpu.sync_copy(ea_g, esum_sh.at[dsti_v], add=True)
            return c1

        lax.fori_loop(0, NGR, group_body, 0)
        return c0

    lax.fori_loop(0, EPW // BE, stage_body, 0)
    plsc.subcore_barrier()

    def db(i, cc):
        rows = pl.ds(base + i * 32, 32)
        orow = pl.ds(cid * NPAD + base + i * 32, 32)
        pltpu.sync_copy(esum_sh.at[rows], esum_out.at[orow])
        return cc

    lax.fori_loop(0, ROWS_PT // 32, db, 0)


def _sc_esum(dst, edge_attr):
    fn = pl.kernel(
        _esum_body,
        out_type=jax.ShapeDtypeStruct((NC * NPAD, 16), F32),
        mesh=_mesh,
        compiler_params=_sc_params,
        scratch_types=[
            pltpu.VMEM((BE,), I32),
            pltpu.VMEM((G,), I32),
            pltpu.VMEM((G, 16), F32),
            pltpu.VMEM((16, 16), F32),
            pltpu.VMEM_SHARED((NPAD, 16), F32),
            pltpu.SemaphoreType.DMA,
        ],
    )
    return fn(dst, edge_attr)


def _mid(big1r, esumr, cat1, vcat, w2cat, b1):
    NB = 1000

    def body(big_ref, es_ref, cat_ref, v_ref, w_ref, bias_ref,
             xp2t_ref, asad2_ref):
        cnt = jnp.maximum(big_ref[0][:, 130:131], 1.0)
        es = es_ref[0] + es_ref[1]
        sumae = jnp.dot(es, v_ref[...],
                        preferred_element_type=F32)
        a_s = cat_ref[:, 256:260]
        a_d = cat_ref[:, 260:264]
        pre = a_s + a_d + sumae[:, 0:4] / cnt
        wl = jnp.exp(jnp.where(pre >= 0, pre, 0.2 * pre))
        hs = []
        for h in range(4):
            xp_h = cat_ref[:, h * 64:(h + 1) * 64]
            bigv = big_ref[h // 2]
            bg = bigv[:, (h % 2) * 64:((h % 2) + 1) * 64]
            den = bigv[:, 128 + (h % 2):129 + (h % 2)]
            wlh = wl[:, h:h + 1]
            hs.append((bg + wlh * xp_h) / (den + wlh + 1e-16))
        hcat = jnp.concatenate(hs, axis=1) + bias_ref[...]
        hrelu = jnp.maximum(hcat, 0.0)
        res = jnp.dot(hrelu, w_ref[...], preferred_element_type=F32)
        ones = jnp.ones((NB, 1), F32)
        zpad = jnp.zeros((NB, 15), F32)
        for t in range(8):
            xp2t_ref[t] = jnp.concatenate(
                [res[:, t * 128:(t + 1) * 128], ones, zpad], axis=1)
        asad2_ref[...] = jnp.concatenate(
            [res[:, 1024:1032], cnt, jnp.zeros((NB, 7), F32)], axis=1)

    return pl.pallas_call(
        body,
        grid=(N // NB,),
        in_specs=[
            pl.BlockSpec((NC, NB, RW), lambda m: (0, m, 0)),
            pl.BlockSpec((NC, NB, 16), lambda m: (0, m, 0)),
            pl.BlockSpec((NB, 272), lambda m: (m, 0)),
            pl.BlockSpec((ED, 8), lambda m: (0, 0)),
            pl.BlockSpec((HID, 1032), lambda m: (0, 0)),
            pl.BlockSpec((1, HID), lambda m: (0, 0)),
        ],
        out_specs=[
            pl.BlockSpec((8, NB, RW), lambda m: (0, m, 0)),
            pl.BlockSpec((NB, 16), lambda m: (m, 0)),
        ],
        out_shape=[
            jax.ShapeDtypeStruct((8, N, RW), F32),
            jax.ShapeDtypeStruct((N, 16), F32),
        ],
    )(big1r, esumr, cat1, vcat, w2cat, b1)


def _post(big2r, esumr, vcat, asad2, xp2t, b2):
    NB = 1000

    def body(big_ref, es_ref, v_ref, asd_ref, xp_ref, bias_ref, o_ref):
        cnt = jnp.maximum(asd_ref[:, 8:9], 1.0)
        es = es_ref[0] + es_ref[1]
        sumae = jnp.dot(es, v_ref[...], preferred_element_type=F32)
        a_s = asd_ref[:, 0:4]
        a_d = asd_ref[:, 4:8]
        pre = a_s + a_d + sumae[:, 4:8] / cnt
        wl = jnp.exp(jnp.where(pre >= 0, pre, 0.2 * pre))
        for c in range(2):
            acc = None
            for h in range(4):
                wlh = wl[:, h:h + 1]
                den = big_ref[2 * h][:, 128:129]
                o_hc = ((big_ref[2 * h + c][:, 0:128] +
                         wlh * xp_ref[2 * h + c][:, 0:128]) /
                        (den + wlh + 1e-16))
                acc = o_hc if acc is None else acc + o_hc
            o_ref[:, c * 128:(c + 1) * 128] = (
                0.25 * acc + bias_ref[:, c * 128:(c + 1) * 128])

    return pl.pallas_call(
        body,
        grid=(N // NB,),
        in_specs=[
            pl.BlockSpec((8, NB, RW), lambda m: (0, m, 0)),
            pl.BlockSpec((NC, NB, 16), lambda m: (0, m, 0)),
            pl.BlockSpec((ED, 8), lambda m: (0, 0)),
            pl.BlockSpec((NB, 16), lambda m: (m, 0)),
            pl.BlockSpec((8, NB, RW), lambda m: (0, m, 0)),
            pl.BlockSpec((1, HID), lambda m: (0, 0)),
        ],
        out_specs=pl.BlockSpec((NB, HID), lambda m: (m, 0)),
        out_shape=jax.ShapeDtypeStruct((N, HID), F32),
    )(big2r, esumr, vcat, asad2, xp2t, b2)


def kernel(x, edge_index, edge_attr, W1, We1, as1, ad1, ae1, b1,
           W2, We2, as2, ad2, ae2, b2):
    src = edge_index[0]
    dst = edge_index[1]

    Ve1 = (We1.reshape(ED, H, C1) * ae1).sum(-1)
    Ve2 = (We2.reshape(ED, H, C2) * ae2).sum(-1)
    vcat = jnp.concatenate([Ve1, Ve2], axis=1)
    vs1 = (W1.reshape(IN, H, C1) * as1).sum(-1)
    vd1 = (W1.reshape(IN, H, C1) * ad1).sum(-1)
    wcat1 = jnp.concatenate(
        [W1, vs1, vd1, jnp.zeros((IN, 8), F32)], axis=1)
    vs2 = (W2.reshape(HID, H, C2) * as2).sum(-1)
    vd2 = (W2.reshape(HID, H, C2) * ad2).sum(-1)
    w2cat = jnp.concatenate([W2, vs2, vd2], axis=1)

    cat1 = _mm(x, wcat1, 1000)
    ae12 = _mm(edge_attr, vcat, 8000)

    ones3 = jnp.ones((N, 3), F32)
    zpad13 = jnp.zeros((N, 13), F32)
    xp1cat = jnp.concatenate([
        jnp.concatenate([cat1[:, :128], ones3, zpad13], axis=1),
        jnp.concatenate([cat1[:, 128:256], ones3, zpad13], axis=1),
    ], axis=0)
    astab1 = jnp.concatenate(
        [cat1[:, 256:264], jnp.zeros((N, 8), F32)], axis=1)

    esum = _sc_esum(dst, edge_attr)
    esumr = esum.reshape(NC, NPAD, 16)[:, :N]
    big1f = _sc_layer1(src, dst, ae12, astab1, xp1cat)
    big1r = big1f.reshape(NC, NPAD, RW)[:, :N]

    xp2t, asad2 = _mid(big1r, esumr, cat1, vcat, w2cat,
                       b1.reshape(1, HID))

    big2f = _sc_layer2(src, dst, ae12, asad2, xp2t.reshape(8 * N, RW))
    big2r = big2f.reshape(H * NC, NPAD, RW)[:, :N]

    return _post(big2r, esumr, vcat, asad2, xp2t, b2.reshape(1, HID))


# speedup vs baseline: 31.4741x; 1.1527x over previous
import functools
import jax
import jax.numpy as jnp
from jax import lax
from jax.experimental import pallas as pl
from jax.experimental.pallas import tpu as pltpu
from jax.experimental.pallas import tpu_sc as plsc

N = 10000
E = 320000
IN = 128
H = 4
HID = 256
ED = 16
C1 = 64
C2 = 256

NC, NS, L = 2, 16, 16
NPAD = 10240
EPT = E // NS
EPW = E // (NC * NS)
BE = 2000
BW = 2000
G = 80
RW = 144
ROWS_PT = NPAD // NS
F32 = jnp.float32
I32 = jnp.int32

_mesh = plsc.VectorSubcoreMesh(core_axis_name="c", subcore_axis_name="s",
                               num_cores=NC, num_subcores=NS)
_sc_params = pltpu.CompilerParams(needs_layout_passes=False,
                                  use_tc_tiling_on_sc=False)


def _mm(a, b, block_m):
    M, K = a.shape
    Nn = b.shape[1]

    def body(a_ref, b_ref, o_ref):
        o_ref[...] = jnp.dot(a_ref[...], b_ref[...],
                             preferred_element_type=F32)

    return pl.pallas_call(
        body,
        grid=(M // block_m,),
        in_specs=[pl.BlockSpec((block_m, K), lambda m: (m, 0)),
                  pl.BlockSpec((K, Nn), lambda m: (0, 0))],
        out_specs=pl.BlockSpec((block_m, Nn), lambda m: (m, 0)),
        out_shape=jax.ShapeDtypeStruct((M, Nn), F32),
    )(a, b)


def _zero_vmem(ref, nrows, width):
    zv = jnp.zeros((L,), F32)

    def zb(r, c):
        for j in range(width // L):
            ref[r, pl.ds(j * L, L)] = zv
        return c

    lax.fori_loop(0, nrows, zb, 0)


def _zero_shared(zbuf, sh, base):
    def zb(i, c):
        pltpu.sync_copy(zbuf, sh.at[pl.ds(base + i * 16, 16)])
        return c

    lax.fori_loop(0, ROWS_PT // 16, zb, 0)


def _edge_weight(asg_v, adg_v, ae_v, rid16, eidx16, h, ae_col):
    a_s = plsc.load_gather(asg_v, [rid16, jnp.full((L,), h, I32)])
    a_d = plsc.load_gather(adg_v, [rid16, jnp.full((L,), 4 + h, I32)])
    a_e = plsc.load_gather(ae_v, [eidx16, jnp.full((L,), ae_col, I32)])
    pre = a_s + a_d + a_e
    return jnp.exp(jnp.where(pre >= 0, pre, F32(0.2) * pre))


def _stage_pipeline(l1, lane, srcs_v, dsts_v, wst_v, bufs, acc_sh,
                    xp_h, off16, ha, hb):
    desc = {}

    def fire(gi):
        b = bufs[gi % 2]
        for k in range(G // L):
            i0 = gi * G + k * L
            s16 = srcs_v[pl.ds(i0, L)]
            b["srco"][pl.ds(k * L, L)] = s16 + off16
            b["dsti"][pl.ds(k * L, L)] = dsts_v[pl.ds(i0, L)]
        desc[(gi, "r")] = pltpu.async_copy(
            xp_h.at[b["srco"]], b["rows"], b["semr"])

    def process(gi):
        b = bufs[gi % 2]
        desc[(gi, "r")].wait()
        rows_v = b["rows"]

        @plsc.parallel_loop(0, G, 1, unroll=4)
        def _(r):
            e16 = jnp.full((L,), gi * G, I32) + r
            wva = plsc.load_gather(wst_v, [jnp.full((L,), ha, I32), e16])
            one = jnp.ones((L,), F32)
            zero = jnp.zeros((L,), F32)
            if l1:
                wvb = plsc.load_gather(
                    wst_v, [jnp.full((L,), hb, I32), e16])
                wmix = jnp.where(
                    lane == 0, wva,
                    jnp.where(lane == 1, wvb,
                              jnp.where(lane == 2, one, zero)))
            else:
                wvb = wva
                wmix = jnp.where(lane == 0, wva, zero)
            for j in range(RW // L):
                wj = wva if j < 4 else (wvb if j < 8 else wmix)
                rows_v[r, pl.ds(j * L, L)] = (
                    rows_v[r, pl.ds(j * L, L)] * wj)

        desc[(gi, "s")] = pltpu.async_copy(
            b["rows"], acc_sh.at[b["dsti"]], b["semw"], add=True)

    def drain(gi):
        desc[(gi, "s")].wait()

    ngr = BE // G
    fire(0)
    fire(1)
    process(0)

    def steady(gi, c):
        return c

    for gi in range(2, ngr):
        drain(gi - 2)
        fire(gi)
        process(gi - 1)
    process(ngr - 1)
    drain(ngr - 2)
    drain(ngr - 1)


def _mk_bufs(args):
    names = ("srco", "dsti", "rows", "semr", "semw")
    a, b = {}, {}
    it = iter(args)
    for n in names:
        a[n] = next(it)
        b[n] = next(it)
    return (a, b)


_BUF_TYPES = [
    pltpu.VMEM((G,), I32), pltpu.VMEM((G,), I32),
    pltpu.VMEM((G,), I32), pltpu.VMEM((G,), I32),
    pltpu.VMEM((G, RW), F32), pltpu.VMEM((G, RW), F32),
    pltpu.SemaphoreType.DMA, pltpu.SemaphoreType.DMA,
    pltpu.SemaphoreType.DMA, pltpu.SemaphoreType.DMA,
]


def _wcalc_body(l1, src_h, dst_h, ae_h, astab_h, ea_h, wtab_out, esum_out,
                srcs_v, dsts_v, ae_v, asg_v, adg_v, ea_g, srci_v, dsti_v,
                wst_v, zesum_v, esum_sh, sem, sem2, sem3):
    cid = lax.axis_index("c")
    sid = lax.axis_index("s")
    wid = cid * NS + sid
    base = sid * ROWS_PT
    lane = lax.iota(I32, L)

    if l1:
        _zero_vmem(zesum_v, 16, 16)
        _zero_shared(zesum_v, esum_sh, base)
        plsc.subcore_barrier()

    def stage_body(s, c0):
        estart = wid * EPW + s * BW
        pltpu.sync_copy(src_h.at[pl.ds(estart, BW)], srcs_v)
        pltpu.sync_copy(dst_h.at[pl.ds(estart, BW)], dsts_v)
        pltpu.sync_copy(ae_h.at[pl.ds(estart, BW)], ae_v)

        def group_body(g, c1):
            for k in range(G // L):
                i0 = g * G + k * L
                srci_v[pl.ds(k * L, L)] = srcs_v[pl.ds(i0, L)]
                dsti_v[pl.ds(k * L, L)] = dsts_v[pl.ds(i0, L)]
            d2 = pltpu.async_copy(astab_h.at[srci_v], asg_v, sem2)
            d3 = pltpu.async_copy(astab_h.at[dsti_v], adg_v, sem3)
            d2.wait()
            d3.wait()
            for k in range(G // L):
                i0 = g * G + k * L
                rid16 = lane + k * L
                eidx16 = lane + i0
                for h in range(4):
                    ae_col = h if l1 else 4 + h
                    wst_v[h, pl.ds(i0, L)] = _edge_weight(
                        asg_v, adg_v, ae_v, rid16, eidx16, h, ae_col)
            if l1:
                pltpu.async_copy(
                    ea_h.at[pl.ds(estart + g * G, G)], ea_g, sem).wait()
                pltpu.sync_copy(ea_g, esum_sh.at[dsti_v], add=True)
            return c1

        lax.fori_loop(0, BW // G, group_body, 0)
        pltpu.sync_copy(wst_v, wtab_out.at[:, pl.ds(estart, BW)])
        return c0

    lax.fori_loop(0, EPW // BW, stage_body, 0)

    if l1:
        plsc.subcore_barrier()

        def db(i, cc):
            rows = pl.ds(base + i * 32, 32)
            orow = pl.ds(cid * NPAD + base + i * 32, 32)
            pltpu.sync_copy(esum_sh.at[rows], esum_out.at[orow])
            return cc

        lax.fori_loop(0, ROWS_PT // 32, db, 0)


def _sc_wcalc(l1, src, dst, ae12, astab, edge_attr):
    out_type = (jax.ShapeDtypeStruct((4, E), F32),
                jax.ShapeDtypeStruct((NC * NPAD, 16), F32))
    fn = pl.kernel(
        functools.partial(_wcalc_body, l1),
        out_type=out_type,
        mesh=_mesh,
        compiler_params=_sc_params,
        scratch_types=[
            pltpu.VMEM((BW,), I32),
            pltpu.VMEM((BW,), I32),
            pltpu.VMEM((BW, 8), F32),
            pltpu.VMEM((G, 16), F32),
            pltpu.VMEM((G, 16), F32),
            pltpu.VMEM((G, 16), F32),
            pltpu.VMEM((G,), I32),
            pltpu.VMEM((G,), I32),
            pltpu.VMEM((4, BW), F32),
            pltpu.VMEM((16, 16), F32),
            pltpu.VMEM_SHARED((NPAD, 16), F32),
            pltpu.SemaphoreType.DMA,
            pltpu.SemaphoreType.DMA,
            pltpu.SemaphoreType.DMA,
        ],
    )
    return fn(src, dst, ae12, astab, edge_attr)


def _sc1_body(src_h, dst_h, wtab_h, xp_h, big_out,
              srcs_v, dsts_v, wst_v, zero_v, acc_sh, *bufargs):
    cid = lax.axis_index("c")
    sid = lax.axis_index("s")
    base = sid * ROWS_PT
    lane = lax.iota(I32, L)
    bufs = _mk_bufs(bufargs)

    _zero_vmem(zero_v, 16, RW)
    _zero_shared(zero_v, acc_sh, base)
    plsc.subcore_barrier()

    ha = cid * 2
    hb = ha + 1
    off16 = jnp.full((L,), cid * N, I32)

    def stage_body(s, c0):
        estart = sid * EPT + s * BE
        pltpu.sync_copy(src_h.at[pl.ds(estart, BE)], srcs_v)
        pltpu.sync_copy(dst_h.at[pl.ds(estart, BE)], dsts_v)
        pltpu.sync_copy(wtab_h.at[:, pl.ds(estart, BE)], wst_v)
        _stage_pipeline(True, lane, srcs_v, dsts_v, wst_v, bufs, acc_sh,
                        xp_h, off16, ha, hb)
        return c0

    lax.fori_loop(0, EPT // BE, stage_body, 0)
    plsc.subcore_barrier()

    def db(i, cc):
        rows = pl.ds(base + i * 32, 32)
        orow = pl.ds(cid * NPAD + base + i * 32, 32)
        pltpu.sync_copy(acc_sh.at[rows], big_out.at[orow])
        return cc

    lax.fori_loop(0, ROWS_PT // 32, db, 0)


def _sc_layer1(src, dst, wtab1, xp1cat):
    fn = pl.kernel(
        _sc1_body,
        out_type=jax.ShapeDtypeStruct((NC * NPAD, RW), F32),
        mesh=_mesh,
        compiler_params=_sc_params,
        scratch_types=[
            pltpu.VMEM((BE,), I32),
            pltpu.VMEM((BE,), I32),
            pltpu.VMEM((4, BE), F32),
            pltpu.VMEM((16, RW), F32),
            pltpu.VMEM_SHARED((NPAD, RW), F32),
        ] + _BUF_TYPES,
    )
    return fn(src, dst, wtab1, xp1cat)


def _sc2_body(src_h, dst_h, wtab_h, xp_h, big_out,
              srcs_v, dsts_v, wst_v, zero_v, acc_sh, *bufargs):
    cid = lax.axis_index("c")
    sid = lax.axis_index("s")
    base = sid * ROWS_PT
    lane = lax.iota(I32, L)
    bufs = _mk_bufs(bufargs)

    _zero_vmem(zero_v, 16, RW)

    def head_body(h, ch):
        _zero_shared(zero_v, acc_sh, base)
        plsc.subcore_barrier()
        off16 = jnp.full((L,), 1, I32) * ((2 * h + cid) * N)

        def stage_body(s, c0):
            estart = sid * EPT + s * BE
            pltpu.sync_copy(src_h.at[pl.ds(estart, BE)], srcs_v)
            pltpu.sync_copy(dst_h.at[pl.ds(estart, BE)], dsts_v)
            pltpu.sync_copy(
                wtab_h.at[pl.ds(h, 1), pl.ds(estart, BE)], wst_v)
            _stage_pipeline(False, lane, srcs_v, dsts_v, wst_v, bufs,
                            acc_sh, xp_h, off16, 0, 0)
            return c0

        lax.fori_loop(0, EPT // BE, stage_body, 0)
        plsc.subcore_barrier()

        def db(i, cc):
            rows = pl.ds(base + i * 32, 32)
            orow = pl.ds((2 * h + cid) * NPAD + base + i * 32, 32)
            pltpu.sync_copy(acc_sh.at[rows], big_out.at[orow])
            return cc

        lax.fori_loop(0, ROWS_PT // 32, db, 0)
        return ch

    lax.fori_loop(0, H, head_body, 0)


def _sc_layer2(src, dst, wtab2, xp2f):
    fn = pl.kernel(
        _sc2_body,
        out_type=jax.ShapeDtypeStruct((H * NC * NPAD, RW), F32),
        mesh=_mesh,
        compiler_params=_sc_params,
        scratch_types=[
            pltpu.VMEM((BE,), I32),
            pltpu.VMEM((BE,), I32),
            pltpu.VMEM((1, BE), F32),
            pltpu.VMEM((16, RW), F32),
            pltpu.VMEM_SHARED((NPAD, RW), F32),
        ] + _BUF_TYPES,
    )
    return fn(src, dst, wtab2, xp2f)


def _mid(big1r, esumr, cat1, vcat, w2cat, b1):
    NB = 1000

    def body(big_ref, es_ref, cat_ref, v_ref, w_ref, bias_ref,
             xp2t_ref, asad2_ref):
        cnt = jnp.maximum(big_ref[0][:, 130:131], 1.0)
        es = es_ref[0] + es_ref[1]
        sumae = jnp.dot(es, v_ref[...],
                        preferred_element_type=F32)
        a_s = cat_ref[:, 256:260]
        a_d = cat_ref[:, 260:264]
        pre = a_s + a_d + sumae[:, 0:4] / cnt
        wl = jnp.exp(jnp.where(pre >= 0, pre, 0.2 * pre))
        hs = []
        for h in range(4):
            xp_h = cat_ref[:, h * 64:(h + 1) * 64]
            bigv = big_ref[h // 2]
            bg = bigv[:, (h % 2) * 64:((h % 2) + 1) * 64]
            den = bigv[:, 128 + (h % 2):129 + (h % 2)]
            wlh = wl[:, h:h + 1]
            hs.append((bg + wlh * xp_h) / (den + wlh + 1e-16))
        hcat = jnp.concatenate(hs, axis=1) + bias_ref[...]
        hrelu = jnp.maximum(hcat, 0.0)
        res = jnp.dot(hrelu, w_ref[...], preferred_element_type=F32)
        ones = jnp.ones((NB, 1), F32)
        zpad = jnp.zeros((NB, 15), F32)
        for t in range(8):
            xp2t_ref[t] = jnp.concatenate(
                [res[:, t * 128:(t + 1) * 128], ones, zpad], axis=1)
        asad2_ref[...] = jnp.concatenate(
            [res[:, 1024:1032], cnt, jnp.zeros((NB, 7), F32)], axis=1)

    return pl.pallas_call(
        body,
        grid=(N // NB,),
        in_specs=[
            pl.BlockSpec((NC, NB, RW), lambda m: (0, m, 0)),
            pl.BlockSpec((NC, NB, 16), lambda m: (0, m, 0)),
            pl.BlockSpec((NB, 272), lambda m: (m, 0)),
            pl.BlockSpec((ED, 8), lambda m: (0, 0)),
            pl.BlockSpec((HID, 1032), lambda m: (0, 0)),
            pl.BlockSpec((1, HID), lambda m: (0, 0)),
        ],
        out_specs=[
            pl.BlockSpec((8, NB, RW), lambda m: (0, m, 0)),
            pl.BlockSpec((NB, 16), lambda m: (m, 0)),
        ],
        out_shape=[
            jax.ShapeDtypeStruct((8, N, RW), F32),
            jax.ShapeDtypeStruct((N, 16), F32),
        ],
    )(big1r, esumr, cat1, vcat, w2cat, b1)


def _post(big2r, esumr, vcat, asad2, xp2t, b2):
    NB = 1000

    def body(big_ref, es_ref, v_ref, asd_ref, xp_ref, bias_ref, o_ref):
        cnt = jnp.maximum(asd_ref[:, 8:9], 1.0)
        es = es_ref[0] + es_ref[1]
        sumae = jnp.dot(es, v_ref[...], preferred_element_type=F32)
        a_s = asd_ref[:, 0:4]
        a_d = asd_ref[:, 4:8]
        pre = a_s + a_d + sumae[:, 4:8] / cnt
        wl = jnp.exp(jnp.where(pre >= 0, pre, 0.2 * pre))
        for c in range(2):
            acc = None
            for h in range(4):
                wlh = wl[:, h:h + 1]
                den = big_ref[2 * h][:, 128:129]
                o_hc = ((big_ref[2 * h + c][:, 0:128] +
                         wlh * xp_ref[2 * h + c][:, 0:128]) /
                        (den + wlh + 1e-16))
                acc = o_hc if acc is None else acc + o_hc
            o_ref[:, c * 128:(c + 1) * 128] = (
                0.25 * acc + bias_ref[:, c * 128:(c + 1) * 128])

    return pl.pallas_call(
        body,
        grid=(N // NB,),
        in_specs=[
            pl.BlockSpec((8, NB, RW), lambda m: (0, m, 0)),
            pl.BlockSpec((NC, NB, 16), lambda m: (0, m, 0)),
            pl.BlockSpec((ED, 8), lambda m: (0, 0)),
            pl.BlockSpec((NB, 16), lambda m: (m, 0)),
            pl.BlockSpec((8, NB, RW), lambda m: (0, m, 0)),
            pl.BlockSpec((1, HID), lambda m: (0, 0)),
        ],
        out_specs=pl.BlockSpec((NB, HID), lambda m: (m, 0)),
        out_shape=jax.ShapeDtypeStruct((N, HID), F32),
    )(big2r, esumr, vcat, asad2, xp2t, b2)


def kernel(x, edge_index, edge_attr, W1, We1, as1, ad1, ae1, b1,
           W2, We2, as2, ad2, ae2, b2):
    src = edge_index[0]
    dst = edge_index[1]

    Ve1 = (We1.reshape(ED, H, C1) * ae1).sum(-1)
    Ve2 = (We2.reshape(ED, H, C2) * ae2).sum(-1)
    vcat = jnp.concatenate([Ve1, Ve2], axis=1)
    vs1 = (W1.reshape(IN, H, C1) * as1).sum(-1)
    vd1 = (W1.reshape(IN, H, C1) * ad1).sum(-1)
    wcat1 = jnp.concatenate(
        [W1, vs1, vd1, jnp.zeros((IN, 8), F32)], axis=1)
    vs2 = (W2.reshape(HID, H, C2) * as2).sum(-1)
    vd2 = (W2.reshape(HID, H, C2) * ad2).sum(-1)
    w2cat = jnp.concatenate([W2, vs2, vd2], axis=1)

    cat1 = _mm(x, wcat1, 1000)
    ae12 = _mm(edge_attr, vcat, 8000)

    ones3 = jnp.ones((N, 3), F32)
    zpad13 = jnp.zeros((N, 13), F32)
    xp1cat = jnp.concatenate([
        jnp.concatenate([cat1[:, :128], ones3, zpad13], axis=1),
        jnp.concatenate([cat1[:, 128:256], ones3, zpad13], axis=1),
    ], axis=0)
    astab1 = jnp.concatenate(
        [cat1[:, 256:264], jnp.zeros((N, 8), F32)], axis=1)

    wtab1, esum = _sc_wcalc(True, src, dst, ae12, astab1, edge_attr)
    esumr = esum.reshape(NC, NPAD, 16)[:, :N]
    big1f = _sc_layer1(src, dst, wtab1, xp1cat)
    big1r = big1f.reshape(NC, NPAD, RW)[:, :N]

    xp2t, asad2 = _mid(big1r, esumr, cat1, vcat, w2cat,
                       b1.reshape(1, HID))

    wtab2, _unused = _sc_wcalc(False, src, dst, ae12, asad2, edge_attr)
    big2f = _sc_layer2(src, dst, wtab2, xp2t.reshape(8 * N, RW))
    big2r = big2f.reshape(H * NC, NPAD, RW)[:, :N]

    return _post(big2r, esumr, vcat, asad2, xp2t, b2.reshape(1, HID))


# speedup vs baseline: 31.6348x; 1.0051x over previous
import functools
import jax
import jax.numpy as jnp
from jax import lax
from jax.experimental import pallas as pl
from jax.experimental.pallas import tpu as pltpu
from jax.experimental.pallas import tpu_sc as plsc

N = 10000
E = 320000
IN = 128
H = 4
HID = 256
ED = 16
C1 = 64
C2 = 256

NC, NS, L = 2, 16, 16
NPAD = 10240
EPT = E // NS
EPW = E // (NC * NS)
BE = 2000
BW = 2000
G = 80
RW = 144
ROWS_PT = NPAD // NS
F32 = jnp.float32
I32 = jnp.int32

_mesh = plsc.VectorSubcoreMesh(core_axis_name="c", subcore_axis_name="s",
                               num_cores=NC, num_subcores=NS)
_sc_params = pltpu.CompilerParams(needs_layout_passes=False,
                                  use_tc_tiling_on_sc=False)


def _mm(a, b, block_m):
    M, K = a.shape
    Nn = b.shape[1]

    def body(a_ref, b_ref, o_ref):
        o_ref[...] = jnp.dot(a_ref[...], b_ref[...],
                             preferred_element_type=F32)

    return pl.pallas_call(
        body,
        grid=(M // block_m,),
        in_specs=[pl.BlockSpec((block_m, K), lambda m: (m, 0)),
                  pl.BlockSpec((K, Nn), lambda m: (0, 0))],
        out_specs=pl.BlockSpec((block_m, Nn), lambda m: (m, 0)),
        out_shape=jax.ShapeDtypeStruct((M, Nn), F32),
    )(a, b)


def _pre(x, wcat1):
    NB = 1000

    def body(a_ref, b_ref, cat_ref, xp_ref, at_ref):
        res = jnp.dot(a_ref[...], b_ref[...], preferred_element_type=F32)
        cat_ref[...] = res
        ones3 = jnp.ones((NB, 3), F32)
        zp13 = jnp.zeros((NB, 13), F32)
        xp_ref[0] = jnp.concatenate([res[:, :128], ones3, zp13], axis=1)
        xp_ref[1] = jnp.concatenate([res[:, 128:256], ones3, zp13], axis=1)
        at_ref[...] = jnp.concatenate(
            [res[:, 256:264], jnp.zeros((NB, 8), F32)], axis=1)

    return pl.pallas_call(
        body,
        grid=(N // NB,),
        in_specs=[pl.BlockSpec((NB, IN), lambda m: (m, 0)),
                  pl.BlockSpec((IN, 272), lambda m: (0, 0))],
        out_specs=[
            pl.BlockSpec((NB, 272), lambda m: (m, 0)),
            pl.BlockSpec((2, NB, RW), lambda m: (0, m, 0)),
            pl.BlockSpec((NB, 16), lambda m: (m, 0)),
        ],
        out_shape=[
            jax.ShapeDtypeStruct((N, 272), F32),
            jax.ShapeDtypeStruct((2, N, RW), F32),
            jax.ShapeDtypeStruct((N, 16), F32),
        ],
    )(x, wcat1)


def _zero_vmem(ref, nrows, width):
    zv = jnp.zeros((L,), F32)

    def zb(r, c):
        for j in range(width // L):
            ref[r, pl.ds(j * L, L)] = zv
        return c

    lax.fori_loop(0, nrows, zb, 0)


def _zero_shared(zbuf, sh, base, zrows=16):
    def zb(i, c):
        pltpu.sync_copy(zbuf, sh.at[pl.ds(base + i * zrows, zrows)])
        return c

    lax.fori_loop(0, ROWS_PT // zrows, zb, 0)


def _edge_weight(asg_v, adg_v, ae_v, rid16, eidx16, h, ae_col):
    a_s = plsc.load_gather(asg_v, [rid16, jnp.full((L,), h, I32)])
    a_d = plsc.load_gather(adg_v, [rid16, jnp.full((L,), 4 + h, I32)])
    a_e = plsc.load_gather(ae_v, [eidx16, jnp.full((L,), ae_col, I32)])
    pre = a_s + a_d + a_e
    return jnp.exp(jnp.where(pre >= 0, pre, F32(0.2) * pre))


def _stage_pipeline(l1, lane, srcs_v, dsts_v, wst_v, bufs, acc_sh,
                    xp_h, off16, ha, hb):
    desc = {}

    def fire(gi):
        b = bufs[gi % 2]
        for k in range(G // L):
            i0 = gi * G + k * L
            s16 = srcs_v[pl.ds(i0, L)]
            b["srco"][pl.ds(k * L, L)] = s16 + off16
            b["dsti"][pl.ds(k * L, L)] = dsts_v[pl.ds(i0, L)]
        desc[(gi, "r")] = pltpu.async_copy(
            xp_h.at[b["srco"]], b["rows"], b["semr"])

    def process(gi):
        b = bufs[gi % 2]
        desc[(gi, "r")].wait()
        rows_v = b["rows"]

        @plsc.parallel_loop(0, G, 1, unroll=4)
        def _(r):
            e16 = jnp.full((L,), gi * G, I32) + r
            wva = plsc.load_gather(wst_v, [jnp.full((L,), ha, I32), e16])
            one = jnp.ones((L,), F32)
            zero = jnp.zeros((L,), F32)
            if l1:
                wvb = plsc.load_gather(
                    wst_v, [jnp.full((L,), hb, I32), e16])
                wmix = jnp.where(
                    lane == 0, wva,
                    jnp.where(lane == 1, wvb,
                              jnp.where(lane == 2, one, zero)))
            else:
                wvb = wva
                wmix = jnp.where(lane == 0, wva, zero)
            for j in range(RW // L):
                wj = wva if j < 4 else (wvb if j < 8 else wmix)
                rows_v[r, pl.ds(j * L, L)] = (
                    rows_v[r, pl.ds(j * L, L)] * wj)

        desc[(gi, "s")] = pltpu.async_copy(
            b["rows"], acc_sh.at[b["dsti"]], b["semw"], add=True)

    def drain(gi):
        desc[(gi, "s")].wait()

    ngr = BE // G
    fire(0)
    fire(1)
    process(0)

    def steady(gi, c):
        return c

    for gi in range(2, ngr):
        drain(gi - 2)
        fire(gi)
        process(gi - 1)
    process(ngr - 1)
    drain(ngr - 2)
    drain(ngr - 1)


def _mk_bufs(args):
    names = ("srco", "dsti", "rows", "semr", "semw")
    a, b = {}, {}
    it = iter(args)
    for n in names:
        a[n] = next(it)
        b[n] = next(it)
    return (a, b)


_BUF_TYPES = [
    pltpu.VMEM((G,), I32), pltpu.VMEM((G,), I32),
    pltpu.VMEM((G,), I32), pltpu.VMEM((G,), I32),
    pltpu.VMEM((G, RW), F32), pltpu.VMEM((G, RW), F32),
    pltpu.SemaphoreType.DMA, pltpu.SemaphoreType.DMA,
    pltpu.SemaphoreType.DMA, pltpu.SemaphoreType.DMA,
]


def _wcalc_body(l1, src_h, dst_h, ae_h, astab_h, ea_h, wtab_out, esum_out,
                srcs_v, dsts_v, ae_v, asg_v, adg_v, ea_g, srci_v, dsti_v,
                wst_v, zesum_v, esum_sh, sem, sem2, sem3):
    cid = lax.axis_index("c")
    sid = lax.axis_index("s")
    wid = cid * NS + sid
    base = sid * ROWS_PT
    lane = lax.iota(I32, L)

    if l1:
        _zero_vmem(zesum_v, 16, 16)
        _zero_shared(zesum_v, esum_sh, base)
        plsc.subcore_barrier()

    def stage_body(s, c0):
        estart = wid * EPW + s * BW
        pltpu.sync_copy(src_h.at[pl.ds(estart, BW)], srcs_v)
        pltpu.sync_copy(dst_h.at[pl.ds(estart, BW)], dsts_v)
        pltpu.sync_copy(ae_h.at[pl.ds(estart, BW)], ae_v)

        def group_body(g, c1):
            for k in range(G // L):
                i0 = g * G + k * L
                srci_v[pl.ds(k * L, L)] = srcs_v[pl.ds(i0, L)]
                dsti_v[pl.ds(k * L, L)] = dsts_v[pl.ds(i0, L)]
            d2 = pltpu.async_copy(astab_h.at[srci_v], asg_v, sem2)
            d3 = pltpu.async_copy(astab_h.at[dsti_v], adg_v, sem3)
            d2.wait()
            d3.wait()
            for k in range(G // L):
                i0 = g * G + k * L
                rid16 = lane + k * L
                eidx16 = lane + i0
                for h in range(4):
                    ae_col = h if l1 else 4 + h
                    wst_v[h, pl.ds(i0, L)] = _edge_weight(
                        asg_v, adg_v, ae_v, rid16, eidx16, h, ae_col)
            if l1:
                pltpu.async_copy(
                    ea_h.at[pl.ds(estart + g * G, G)], ea_g, sem).wait()
                pltpu.sync_copy(ea_g, esum_sh.at[dsti_v], add=True)
            return c1

        lax.fori_loop(0, BW // G, group_body, 0)
        pltpu.sync_copy(wst_v, wtab_out.at[:, pl.ds(estart, BW)])
        return c0

    lax.fori_loop(0, EPW // BW, stage_body, 0)

    if l1:
        plsc.subcore_barrier()

        def db(i, cc):
            rows = pl.ds(base + i * 32, 32)
            orow = pl.ds(cid * NPAD + base + i * 32, 32)
            pltpu.sync_copy(esum_sh.at[rows], esum_out.at[orow])
            return cc

        lax.fori_loop(0, ROWS_PT // 32, db, 0)


def _sc_wcalc(l1, src, dst, ae12, astab, edge_attr):
    out_type = (jax.ShapeDtypeStruct((4, E), F32),
                jax.ShapeDtypeStruct((NC * NPAD, 16), F32))
    fn = pl.kernel(
        functools.partial(_wcalc_body, l1),
        out_type=out_type,
        mesh=_mesh,
        compiler_params=_sc_params,
        scratch_types=[
            pltpu.VMEM((BW,), I32),
            pltpu.VMEM((BW,), I32),
            pltpu.VMEM((BW, 8), F32),
            pltpu.VMEM((G, 16), F32),
            pltpu.VMEM((G, 16), F32),
            pltpu.VMEM((G, 16), F32),
            pltpu.VMEM((G,), I32),
            pltpu.VMEM((G,), I32),
            pltpu.VMEM((4, BW), F32),
            pltpu.VMEM((16, 16), F32),
            pltpu.VMEM_SHARED((NPAD, 16), F32),
            pltpu.SemaphoreType.DMA,
            pltpu.SemaphoreType.DMA,
            pltpu.SemaphoreType.DMA,
        ],
    )
    return fn(src, dst, ae12, astab, edge_attr)


def _sc1_body(src_h, dst_h, wtab_h, xp_h, big_out,
              srcs_v, dsts_v, wst_v, zero_v, acc_sh, *bufargs):
    cid = lax.axis_index("c")
    sid = lax.axis_index("s")
    base = sid * ROWS_PT
    lane = lax.iota(I32, L)
    bufs = _mk_bufs(bufargs)

    _zero_vmem(zero_v, 16, RW)
    _zero_shared(zero_v, acc_sh, base)
    plsc.subcore_barrier()

    ha = cid * 2
    hb = ha + 1
    off16 = jnp.full((L,), cid * N, I32)

    def stage_body(s, c0):
        estart = sid * EPT + s * BE
        pltpu.sync_copy(src_h.at[pl.ds(estart, BE)], srcs_v)
        pltpu.sync_copy(dst_h.at[pl.ds(estart, BE)], dsts_v)
        pltpu.sync_copy(wtab_h.at[:, pl.ds(estart, BE)], wst_v)
        _stage_pipeline(True, lane, srcs_v, dsts_v, wst_v, bufs, acc_sh,
                        xp_h, off16, ha, hb)
        return c0

    lax.fori_loop(0, EPT // BE, stage_body, 0)
    plsc.subcore_barrier()

    def db(i, cc):
        rows = pl.ds(base + i * 32, 32)
        orow = pl.ds(cid * NPAD + base + i * 32, 32)
        pltpu.sync_copy(acc_sh.at[rows], big_out.at[orow])
        return cc

    lax.fori_loop(0, ROWS_PT // 32, db, 0)


def _sc_layer1(src, dst, wtab1, xp1cat):
    fn = pl.kernel(
        _sc1_body,
        out_type=jax.ShapeDtypeStruct((NC * NPAD, RW), F32),
        mesh=_mesh,
        compiler_params=_sc_params,
        scratch_types=[
            pltpu.VMEM((BE,), I32),
            pltpu.VMEM((BE,), I32),
            pltpu.VMEM((4, BE), F32),
            pltpu.VMEM((16, RW), F32),
            pltpu.VMEM_SHARED((NPAD, RW), F32),
        ] + _BUF_TYPES,
    )
    return fn(src, dst, wtab1, xp1cat)


def _sc2_body(src_h, dst_h, wtab_h, xp_h, big_out,
              srcs_v, dsts_v, wst_v, zero_v, acc_sh, *bufargs):
    cid = lax.axis_index("c")
    sid = lax.axis_index("s")
    base = sid * ROWS_PT
    lane = lax.iota(I32, L)
    bufs = _mk_bufs(bufargs)

    _zero_vmem(zero_v, 64, RW)

    def head_body(h, ch):
        _zero_shared(zero_v, acc_sh, base, 64)
        plsc.subcore_barrier()
        off16 = jnp.full((L,), 1, I32) * ((2 * h + cid) * N)

        def stage_body(s, c0):
            estart = sid * EPT + s * BE
            pltpu.sync_copy(src_h.at[pl.ds(estart, BE)], srcs_v)
            pltpu.sync_copy(dst_h.at[pl.ds(estart, BE)], dsts_v)
            pltpu.sync_copy(
                wtab_h.at[pl.ds(h, 1), pl.ds(estart, BE)], wst_v)
            _stage_pipeline(False, lane, srcs_v, dsts_v, wst_v, bufs,
                            acc_sh, xp_h, off16, 0, 0)
            return c0

        lax.fori_loop(0, EPT // BE, stage_body, 0)
        plsc.subcore_barrier()

        def db(i, cc):
            rows = pl.ds(base + i * 32, 32)
            orow = pl.ds((2 * h + cid) * NPAD + base + i * 32, 32)
            pltpu.sync_copy(acc_sh.at[rows], big_out.at[orow])
            return cc

        lax.fori_loop(0, ROWS_PT // 32, db, 0)
        return ch

    lax.fori_loop(0, H, head_body, 0)


def _sc_layer2(src, dst, wtab2, xp2f):
    fn = pl.kernel(
        _sc2_body,
        out_type=jax.ShapeDtypeStruct((H * NC * NPAD, RW), F32),
        mesh=_mesh,
        compiler_params=_sc_params,
        scratch_types=[
            pltpu.VMEM((BE,), I32),
            pltpu.VMEM((BE,), I32),
            pltpu.VMEM((1, BE), F32),
            pltpu.VMEM((64, RW), F32),
            pltpu.VMEM_SHARED((NPAD, RW), F32),
        ] + _BUF_TYPES,
    )
    return fn(src, dst, wtab2, xp2f)


def _mid(big1r, esumr, cat1, vcat, w2cat, b1):
    NB = 1000

    def body(big_ref, es_ref, cat_ref, v_ref, w_ref, bias_ref,
             xp2t_ref, asad2_ref):
        cnt = jnp.maximum(big_ref[0][:, 130:131], 1.0)
        es = es_ref[0] + es_ref[1]
        sumae = jnp.dot(es, v_ref[...],
                        preferred_element_type=F32)
        a_s = cat_ref[:, 256:260]
        a_d = cat_ref[:, 260:264]
        pre = a_s + a_d + sumae[:, 0:4] / cnt
        wl = jnp.exp(jnp.where(pre >= 0, pre, 0.2 * pre))
        hs = []
        for h in range(4):
            xp_h = cat_ref[:, h * 64:(h + 1) * 64]
            bigv = big_ref[h // 2]
            bg = bigv[:, (h % 2) * 64:((h % 2) + 1) * 64]
            den = bigv[:, 128 + (h % 2):129 + (h % 2)]
            wlh = wl[:, h:h + 1]
            hs.append((bg + wlh * xp_h) / (den + wlh + 1e-16))
        hcat = jnp.concatenate(hs, axis=1) + bias_ref[...]
        hrelu = jnp.maximum(hcat, 0.0)
        res = jnp.dot(hrelu, w_ref[...], preferred_element_type=F32)
        ones = jnp.ones((NB, 1), F32)
        zpad = jnp.zeros((NB, 15), F32)
        for t in range(8):
            xp2t_ref[t] = jnp.concatenate(
                [res[:, t * 128:(t + 1) * 128], ones, zpad], axis=1)
        asad2_ref[...] = jnp.concatenate(
            [res[:, 1024:1032], cnt, jnp.zeros((NB, 7), F32)], axis=1)

    return pl.pallas_call(
        body,
        grid=(N // NB,),
        in_specs=[
            pl.BlockSpec((NC, NB, RW), lambda m: (0, m, 0)),
            pl.BlockSpec((NC, NB, 16), lambda m: (0, m, 0)),
            pl.BlockSpec((NB, 272), lambda m: (m, 0)),
            pl.BlockSpec((ED, 8), lambda m: (0, 0)),
            pl.BlockSpec((HID, 1032), lambda m: (0, 0)),
            pl.BlockSpec((1, HID), lambda m: (0, 0)),
        ],
        out_specs=[
            pl.BlockSpec((8, NB, RW), lambda m: (0, m, 0)),
            pl.BlockSpec((NB, 16), lambda m: (m, 0)),
        ],
        out_shape=[
            jax.ShapeDtypeStruct((8, N, RW), F32),
            jax.ShapeDtypeStruct((N, 16), F32),
        ],
    )(big1r, esumr, cat1, vcat, w2cat, b1)


def _post(big2r, esumr, vcat, asad2, xp2t, b2):
    NB = 1000

    def body(big_ref, es_ref, v_ref, asd_ref, xp_ref, bias_ref, o_ref):
        cnt = jnp.maximum(asd_ref[:, 8:9], 1.0)
        es = es_ref[0] + es_ref[1]
        sumae = jnp.dot(es, v_ref[...], preferred_element_type=F32)
        a_s = asd_ref[:, 0:4]
        a_d = asd_ref[:, 4:8]
        pre = a_s + a_d + sumae[:, 4:8] / cnt
        wl = jnp.exp(jnp.where(pre >= 0, pre, 0.2 * pre))
        for c in range(2):
            acc = None
            for h in range(4):
                wlh = wl[:, h:h + 1]
                den = big_ref[2 * h][:, 128:129]
                o_hc = ((big_ref[2 * h + c][:, 0:128] +
                         wlh * xp_ref[2 * h + c][:, 0:128]) /
                        (den + wlh + 1e-16))
                acc = o_hc if acc is None else acc + o_hc
            o_ref[:, c * 128:(c + 1) * 128] = (
                0.25 * acc + bias_ref[:, c * 128:(c + 1) * 128])

    return pl.pallas_call(
        body,
        grid=(N // NB,),
        in_specs=[
            pl.BlockSpec((8, NB, RW), lambda m: (0, m, 0)),
            pl.BlockSpec((NC, NB, 16), lambda m: (0, m, 0)),
            pl.BlockSpec((ED, 8), lambda m: (0, 0)),
            pl.BlockSpec((NB, 16), lambda m: (m, 0)),
            pl.BlockSpec((8, NB, RW), lambda m: (0, m, 0)),
            pl.BlockSpec((1, HID), lambda m: (0, 0)),
        ],
        out_specs=pl.BlockSpec((NB, HID), lambda m: (m, 0)),
        out_shape=jax.ShapeDtypeStruct((N, HID), F32),
    )(big2r, esumr, vcat, asad2, xp2t, b2)


def kernel(x, edge_index, edge_attr, W1, We1, as1, ad1, ae1, b1,
           W2, We2, as2, ad2, ae2, b2):
    src = edge_index[0]
    dst = edge_index[1]

    Ve1 = (We1.reshape(ED, H, C1) * ae1).sum(-1)
    Ve2 = (We2.reshape(ED, H, C2) * ae2).sum(-1)
    vcat = jnp.concatenate([Ve1, Ve2], axis=1)
    vs1 = (W1.reshape(IN, H, C1) * as1).sum(-1)
    vd1 = (W1.reshape(IN, H, C1) * ad1).sum(-1)
    wcat1 = jnp.concatenate(
        [W1, vs1, vd1, jnp.zeros((IN, 8), F32)], axis=1)
    vs2 = (W2.reshape(HID, H, C2) * as2).sum(-1)
    vd2 = (W2.reshape(HID, H, C2) * ad2).sum(-1)
    w2cat = jnp.concatenate([W2, vs2, vd2], axis=1)

    cat1, xp1x, astab1 = _pre(x, wcat1)
    ae12 = _mm(edge_attr, vcat, 8000)
    xp1cat = xp1x.reshape(2 * N, RW)

    wtab1, esum = _sc_wcalc(True, src, dst, ae12, astab1, edge_attr)
    esumr = esum.reshape(NC, NPAD, 16)[:, :N]
    big1f = _sc_layer1(src, dst, wtab1, xp1cat)
    big1r = big1f.reshape(NC, NPAD, RW)[:, :N]

    xp2t, asad2 = _mid(big1r, esumr, cat1, vcat, w2cat,
                       b1.reshape(1, HID))

    wtab2, _unused = _sc_wcalc(False, src, dst, ae12, asad2, edge_attr)
    big2f = _sc_layer2(src, dst, wtab2, xp2t.reshape(8 * N, RW))
    big2r = big2f.reshape(H * NC, NPAD, RW)[:, :N]

    return _post(big2r, esumr, vcat, asad2, xp2t, b2.reshape(1, HID))


# speedup vs baseline: 33.0813x; 1.0457x over previous
import functools
import jax
import jax.numpy as jnp
from jax import lax
from jax.experimental import pallas as pl
from jax.experimental.pallas import tpu as pltpu
from jax.experimental.pallas import tpu_sc as plsc

N = 10000
E = 320000
IN = 128
H = 4
HID = 256
ED = 16
C1 = 64
C2 = 256

NC, NS, L = 2, 16, 16
NPAD = 10240
EPT = E // NS
EPW = E // (NC * NS)
BE = 2000
BW = 400
G = 80
RW = 144
ROWS_PT = NPAD // NS
F32 = jnp.float32
I32 = jnp.int32

_mesh = plsc.VectorSubcoreMesh(core_axis_name="c", subcore_axis_name="s",
                               num_cores=NC, num_subcores=NS)
_sc_params = pltpu.CompilerParams(needs_layout_passes=False,
                                  use_tc_tiling_on_sc=False)


def _mm(a, b, block_m):
    M, K = a.shape
    Nn = b.shape[1]

    def body(a_ref, b_ref, o_ref):
        o_ref[...] = jnp.dot(a_ref[...], b_ref[...],
                             preferred_element_type=F32)

    return pl.pallas_call(
        body,
        grid=(M // block_m,),
        in_specs=[pl.BlockSpec((block_m, K), lambda m: (m, 0)),
                  pl.BlockSpec((K, Nn), lambda m: (0, 0))],
        out_specs=pl.BlockSpec((block_m, Nn), lambda m: (m, 0)),
        out_shape=jax.ShapeDtypeStruct((M, Nn), F32),
    )(a, b)


def _pre(x, wcat1):
    NB = 1000

    def body(a_ref, b_ref, cat_ref, xp_ref, at_ref):
        res = jnp.dot(a_ref[...], b_ref[...], preferred_element_type=F32)
        cat_ref[...] = res
        ones3 = jnp.ones((NB, 3), F32)
        zp13 = jnp.zeros((NB, 13), F32)
        xp_ref[0] = jnp.concatenate([res[:, :128], ones3, zp13], axis=1)
        xp_ref[1] = jnp.concatenate([res[:, 128:256], ones3, zp13], axis=1)
        at_ref[...] = jnp.concatenate(
            [res[:, 256:264], jnp.zeros((NB, 8), F32)], axis=1)

    return pl.pallas_call(
        body,
        grid=(N // NB,),
        in_specs=[pl.BlockSpec((NB, IN), lambda m: (m, 0)),
                  pl.BlockSpec((IN, 272), lambda m: (0, 0))],
        out_specs=[
            pl.BlockSpec((NB, 272), lambda m: (m, 0)),
            pl.BlockSpec((2, NB, RW), lambda m: (0, m, 0)),
            pl.BlockSpec((NB, 16), lambda m: (m, 0)),
        ],
        out_shape=[
            jax.ShapeDtypeStruct((N, 272), F32),
            jax.ShapeDtypeStruct((2, N, RW), F32),
            jax.ShapeDtypeStruct((N, 16), F32),
        ],
    )(x, wcat1)


def _zero_vmem(ref, nrows, width):
    zv = jnp.zeros((L,), F32)

    def zb(r, c):
        for j in range(width // L):
            ref[r, pl.ds(j * L, L)] = zv
        return c

    lax.fori_loop(0, nrows, zb, 0)


def _zero_shared(zbuf, sh, base, zrows=16):
    def zb(i, c):
        pltpu.sync_copy(zbuf, sh.at[pl.ds(base + i * zrows, zrows)])
        return c

    lax.fori_loop(0, ROWS_PT // zrows, zb, 0)


def _edge_weight(asg_v, adg_v, ae_v, rid16, eidx16, h, ae_col):
    a_s = plsc.load_gather(asg_v, [rid16, jnp.full((L,), h, I32)])
    a_d = plsc.load_gather(adg_v, [rid16, jnp.full((L,), 4 + h, I32)])
    a_e = plsc.load_gather(ae_v, [eidx16, jnp.full((L,), ae_col, I32)])
    pre = a_s + a_d + a_e
    return jnp.exp(jnp.where(pre >= 0, pre, F32(0.2) * pre))


def _stage_pipeline(l1, lane, srcs_v, dsts_v, wst_v, bufs, acc_sh,
                    xp_h, off16, ha, hb):
    desc = {}

    def fire(gi):
        b = bufs[gi % 2]
        for k in range(G // L):
            i0 = gi * G + k * L
            s16 = srcs_v[pl.ds(i0, L)]
            b["srco"][pl.ds(k * L, L)] = s16 + off16
            b["dsti"][pl.ds(k * L, L)] = dsts_v[pl.ds(i0, L)]
        desc[(gi, "r")] = pltpu.async_copy(
            xp_h.at[b["srco"]], b["rows"], b["semr"])

    def process(gi):
        b = bufs[gi % 2]
        desc[(gi, "r")].wait()
        rows_v = b["rows"]

        @plsc.parallel_loop(0, G, 1, unroll=4)
        def _(r):
            e16 = jnp.full((L,), gi * G, I32) + r
            wva = plsc.load_gather(wst_v, [jnp.full((L,), ha, I32), e16])
            one = jnp.ones((L,), F32)
            zero = jnp.zeros((L,), F32)
            if l1:
                wvb = plsc.load_gather(
                    wst_v, [jnp.full((L,), hb, I32), e16])
                wmix = jnp.where(
                    lane == 0, wva,
                    jnp.where(lane == 1, wvb,
                              jnp.where(lane == 2, one, zero)))
            else:
                wvb = wva
                wmix = jnp.where(lane == 0, wva, zero)
            for j in range(RW // L):
                wj = wva if j < 4 else (wvb if j < 8 else wmix)
                rows_v[r, pl.ds(j * L, L)] = (
                    rows_v[r, pl.ds(j * L, L)] * wj)

        desc[(gi, "s")] = pltpu.async_copy(
            b["rows"], acc_sh.at[b["dsti"]], b["semw"], add=True)

    def drain(gi):
        desc[(gi, "s")].wait()

    ngr = BE // G
    fire(0)
    fire(1)
    process(0)

    def steady(gi, c):
        return c

    for gi in range(2, ngr):
        drain(gi - 2)
        fire(gi)
        process(gi - 1)
    process(ngr - 1)
    drain(ngr - 2)
    drain(ngr - 1)


def _mk_bufs(args):
    names = ("srco", "dsti", "rows", "semr", "semw")
    a, b = {}, {}
    it = iter(args)
    for n in names:
        a[n] = next(it)
        b[n] = next(it)
    return (a, b)


_BUF_TYPES = [
    pltpu.VMEM((G,), I32), pltpu.VMEM((G,), I32),
    pltpu.VMEM((G,), I32), pltpu.VMEM((G,), I32),
    pltpu.VMEM((G, RW), F32), pltpu.VMEM((G, RW), F32),
    pltpu.SemaphoreType.DMA, pltpu.SemaphoreType.DMA,
    pltpu.SemaphoreType.DMA, pltpu.SemaphoreType.DMA,
]


def _wcalc_body(l1, src_h, dst_h, ae_h, astab_h, ea_h, wtab_out, esum_out,
                srcs_v, dsts_v, ae_v, wst_v, zesum_v, esum_sh, *wbufargs):
    cid = lax.axis_index("c")
    sid = lax.axis_index("s")
    wid = cid * NS + sid
    base = sid * ROWS_PT
    lane = lax.iota(I32, L)

    names = ("srci", "dsti", "asg", "adg", "eag", "sema", "semd", "seme")
    wb = ({}, {})
    it = iter(wbufargs)
    for n in names:
        wb[0][n] = next(it)
        wb[1][n] = next(it)

    if l1:
        _zero_vmem(zesum_v, 16, 16)
        _zero_shared(zesum_v, esum_sh, base)
        plsc.subcore_barrier()

    def stage_body(s, c0):
        estart = wid * EPW + s * BW
        pltpu.sync_copy(src_h.at[pl.ds(estart, BW)], srcs_v)
        pltpu.sync_copy(dst_h.at[pl.ds(estart, BW)], dsts_v)
        pltpu.sync_copy(ae_h.at[pl.ds(estart, BW)], ae_v)
        desc = {}

        def fire(gi):
            b = wb[gi % 2]
            for k in range(G // L):
                i0 = gi * G + k * L
                b["srci"][pl.ds(k * L, L)] = srcs_v[pl.ds(i0, L)]
                b["dsti"][pl.ds(k * L, L)] = dsts_v[pl.ds(i0, L)]
            desc[(gi, "a")] = pltpu.async_copy(
                astab_h.at[b["srci"]], b["asg"], b["sema"])
            desc[(gi, "d")] = pltpu.async_copy(
                astab_h.at[b["dsti"]], b["adg"], b["semd"])
            if l1:
                desc[(gi, "e")] = pltpu.async_copy(
                    ea_h.at[pl.ds(estart + gi * G, G)], b["eag"],
                    b["seme"])

        def process(gi):
            b = wb[gi % 2]
            desc[(gi, "a")].wait()
            desc[(gi, "d")].wait()
            for k in range(G // L):
                i0 = gi * G + k * L
                rid16 = lane + k * L
                eidx16 = lane + i0
                for h in range(4):
                    ae_col = h if l1 else 4 + h
                    wst_v[h, pl.ds(i0, L)] = _edge_weight(
                        b["asg"], b["adg"], ae_v, rid16, eidx16, h, ae_col)
            if l1:
                desc[(gi, "e")].wait()
                pltpu.sync_copy(b["eag"], esum_sh.at[b["dsti"]], add=True)

        ngr = BW // G
        fire(0)
        fire(1)
        process(0)
        for gi in range(2, ngr):
            fire(gi)
            process(gi - 1)
        process(ngr - 1)

        pltpu.sync_copy(wst_v, wtab_out.at[:, pl.ds(estart, BW)])
        return c0

    lax.fori_loop(0, EPW // BW, stage_body, 0)

    if l1:
        plsc.subcore_barrier()

        def db(i, cc):
            rows = pl.ds(base + i * 32, 32)
            orow = pl.ds(cid * NPAD + base + i * 32, 32)
            pltpu.sync_copy(esum_sh.at[rows], esum_out.at[orow])
            return cc

        lax.fori_loop(0, ROWS_PT // 32, db, 0)


def _sc_wcalc(l1, src, dst, ae12, astab, edge_attr):
    out_type = (jax.ShapeDtypeStruct((4, E), F32),
                jax.ShapeDtypeStruct((NC * NPAD, 16), F32))
    fn = pl.kernel(
        functools.partial(_wcalc_body, l1),
        out_type=out_type,
        mesh=_mesh,
        compiler_params=_sc_params,
        scratch_types=[
            pltpu.VMEM((BW,), I32),
            pltpu.VMEM((BW,), I32),
            pltpu.VMEM((BW, 8), F32),
            pltpu.VMEM((4, BW), F32),
            pltpu.VMEM((16, 16), F32),
            pltpu.VMEM_SHARED((NPAD, 16), F32),
            pltpu.VMEM((G,), I32), pltpu.VMEM((G,), I32),
            pltpu.VMEM((G,), I32), pltpu.VMEM((G,), I32),
            pltpu.VMEM((G, 16), F32), pltpu.VMEM((G, 16), F32),
            pltpu.VMEM((G, 16), F32), pltpu.VMEM((G, 16), F32),
            pltpu.VMEM((G, 16), F32), pltpu.VMEM((G, 16), F32),
            pltpu.SemaphoreType.DMA, pltpu.SemaphoreType.DMA,
            pltpu.SemaphoreType.DMA, pltpu.SemaphoreType.DMA,
            pltpu.SemaphoreType.DMA, pltpu.SemaphoreType.DMA,
        ],
    )
    return fn(src, dst, ae12, astab, edge_attr)


def _sc1_body(src_h, dst_h, wtab_h, xp_h, big_out,
              srcs_v, dsts_v, wst_v, zero_v, acc_sh, *bufargs):
    cid = lax.axis_index("c")
    sid = lax.axis_index("s")
    base = sid * ROWS_PT
    lane = lax.iota(I32, L)
    bufs = _mk_bufs(bufargs)

    _zero_vmem(zero_v, 16, RW)
    _zero_shared(zero_v, acc_sh, base)
    plsc.subcore_barrier()

    ha = cid * 2
    hb = ha + 1
    off16 = jnp.full((L,), cid * N, I32)

    def stage_body(s, c0):
        estart = sid * EPT + s * BE
        pltpu.sync_copy(src_h.at[pl.ds(estart, BE)], srcs_v)
        pltpu.sync_copy(dst_h.at[pl.ds(estart, BE)], dsts_v)
        pltpu.sync_copy(wtab_h.at[:, pl.ds(estart, BE)], wst_v)
        _stage_pipeline(True, lane, srcs_v, dsts_v, wst_v, bufs, acc_sh,
                        xp_h, off16, ha, hb)
        return c0

    lax.fori_loop(0, EPT // BE, stage_body, 0)
    plsc.subcore_barrier()

    def db(i, cc):
        rows = pl.ds(base + i * 32, 32)
        orow = pl.ds(cid * NPAD + base + i * 32, 32)
        pltpu.sync_copy(acc_sh.at[rows], big_out.at[orow])
        return cc

    lax.fori_loop(0, ROWS_PT // 32, db, 0)


def _sc_layer1(src, dst, wtab1, xp1cat):
    fn = pl.kernel(
        _sc1_body,
        out_type=jax.ShapeDtypeStruct((NC * NPAD, RW), F32),
        mesh=_mesh,
        compiler_params=_sc_params,
        scratch_types=[
            pltpu.VMEM((BE,), I32),
            pltpu.VMEM((BE,), I32),
            pltpu.VMEM((4, BE), F32),
            pltpu.VMEM((16, RW), F32),
            pltpu.VMEM_SHARED((NPAD, RW), F32),
        ] + _BUF_TYPES,
    )
    return fn(src, dst, wtab1, xp1cat)


def _sc2_body(src_h, dst_h, wtab_h, xp_h, big_out,
              srcs_v, dsts_v, wst_v, zero_v, acc_sh, *bufargs):
    cid = lax.axis_index("c")
    sid = lax.axis_index("s")
    base = sid * ROWS_PT
    lane = lax.iota(I32, L)
    bufs = _mk_bufs(bufargs)

    _zero_vmem(zero_v, 64, RW)

    def head_body(h, ch):
        _zero_shared(zero_v, acc_sh, base, 64)
        plsc.subcore_barrier()
        off16 = jnp.full((L,), 1, I32) * ((2 * h + cid) * N)

        def stage_body(s, c0):
            estart = sid * EPT + s * BE
            pltpu.sync_copy(src_h.at[pl.ds(estart, BE)], srcs_v)
            pltpu.sync_copy(dst_h.at[pl.ds(estart, BE)], dsts_v)
            pltpu.sync_copy(
                wtab_h.at[pl.ds(h, 1), pl.ds(estart, BE)], wst_v)
            _stage_pipeline(False, lane, srcs_v, dsts_v, wst_v, bufs,
                            acc_sh, xp_h, off16, 0, 0)
            return c0

        lax.fori_loop(0, EPT // BE, stage_body, 0)
        plsc.subcore_barrier()

        def db(i, cc):
            rows = pl.ds(base + i * 32, 32)
            orow = pl.ds((2 * h + cid) * NPAD + base + i * 32, 32)
            pltpu.sync_copy(acc_sh.at[rows], big_out.at[orow])
            return cc

        lax.fori_loop(0, ROWS_PT // 32, db, 0)
        return ch

    lax.fori_loop(0, H, head_body, 0)


def _sc_layer2(src, dst, wtab2, xp2f):
    fn = pl.kernel(
        _sc2_body,
        out_type=jax.ShapeDtypeStruct((H * NC * NPAD, RW), F32),
        mesh=_mesh,
        compiler_params=_sc_params,
        scratch_types=[
            pltpu.VMEM((BE,), I32),
            pltpu.VMEM((BE,), I32),
            pltpu.VMEM((1, BE), F32),
            pltpu.VMEM((64, RW), F32),
            pltpu.VMEM_SHARED((NPAD, RW), F32),
        ] + _BUF_TYPES,
    )
    return fn(src, dst, wtab2, xp2f)


def _mid(big1r, esumr, cat1, vcat, w2cat, b1):
    NB = 1000

    def body(big_ref, es_ref, cat_ref, v_ref, w_ref, bias_ref,
             xp2t_ref, asad2_ref):
        cnt = jnp.maximum(big_ref[0][:, 130:131], 1.0)
        es = es_ref[0] + es_ref[1]
        sumae = jnp.dot(es, v_ref[...],
                        preferred_element_type=F32)
        a_s = cat_ref[:, 256:260]
        a_d = cat_ref[:, 260:264]
        pre = a_s + a_d + sumae[:, 0:4] / cnt
        wl = jnp.exp(jnp.where(pre >= 0, pre, 0.2 * pre))
        hs = []
        for h in range(4):
            xp_h = cat_ref[:, h * 64:(h + 1) * 64]
            bigv = big_ref[h // 2]
            bg = bigv[:, (h % 2) * 64:((h % 2) + 1) * 64]
            den = bigv[:, 128 + (h % 2):129 + (h % 2)]
            wlh = wl[:, h:h + 1]
            hs.append((bg + wlh * xp_h) / (den + wlh + 1e-16))
        hcat = jnp.concatenate(hs, axis=1) + bias_ref[...]
        hrelu = jnp.maximum(hcat, 0.0)
        res = jnp.dot(hrelu, w_ref[...], preferred_element_type=F32)
        ones = jnp.ones((NB, 1), F32)
        zpad = jnp.zeros((NB, 15), F32)
        for t in range(8):
            xp2t_ref[t] = jnp.concatenate(
                [res[:, t * 128:(t + 1) * 128], ones, zpad], axis=1)
        asad2_ref[...] = jnp.concatenate(
            [res[:, 1024:1032], cnt, jnp.zeros((NB, 7), F32)], axis=1)

    return pl.pallas_call(
        body,
        grid=(N // NB,),
        in_specs=[
            pl.BlockSpec((NC, NB, RW), lambda m: (0, m, 0)),
            pl.BlockSpec((NC, NB, 16), lambda m: (0, m, 0)),
            pl.BlockSpec((NB, 272), lambda m: (m, 0)),
            pl.BlockSpec((ED, 8), lambda m: (0, 0)),
            pl.BlockSpec((HID, 1032), lambda m: (0, 0)),
            pl.BlockSpec((1, HID), lambda m: (0, 0)),
        ],
        out_specs=[
            pl.BlockSpec((8, NB, RW), lambda m: (0, m, 0)),
            pl.BlockSpec((NB, 16), lambda m: (m, 0)),
        ],
        out_shape=[
            jax.ShapeDtypeStruct((8, N, RW), F32),
            jax.ShapeDtypeStruct((N, 16), F32),
        ],
    )(big1r, esumr, cat1, vcat, w2cat, b1)


def _post(big2r, esumr, vcat, asad2, xp2t, b2):
    NB = 1000

    def body(big_ref, es_ref, v_ref, asd_ref, xp_ref, bias_ref, o_ref):
        cnt = jnp.maximum(asd_ref[:, 8:9], 1.0)
        es = es_ref[0] + es_ref[1]
        sumae = jnp.dot(es, v_ref[...], preferred_element_type=F32)
        a_s = asd_ref[:, 0:4]
        a_d = asd_ref[:, 4:8]
        pre = a_s + a_d + sumae[:, 4:8] / cnt
        wl = jnp.exp(jnp.where(pre >= 0, pre, 0.2 * pre))
        for c in range(2):
            acc = None
            for h in range(4):
                wlh = wl[:, h:h + 1]
                den = big_ref[2 * h][:, 128:129]
                o_hc = ((big_ref[2 * h + c][:, 0:128] +
                         wlh * xp_ref[2 * h + c][:, 0:128]) /
                        (den + wlh + 1e-16))
                acc = o_hc if acc is None else acc + o_hc
            o_ref[:, c * 128:(c + 1) * 128] = (
                0.25 * acc + bias_ref[:, c * 128:(c + 1) * 128])

    return pl.pallas_call(
        body,
        grid=(N // NB,),
        in_specs=[
            pl.BlockSpec((8, NB, RW), lambda m: (0, m, 0)),
            pl.BlockSpec((NC, NB, 16), lambda m: (0, m, 0)),
            pl.BlockSpec((ED, 8), lambda m: (0, 0)),
            pl.BlockSpec((NB, 16), lambda m: (m, 0)),
            pl.BlockSpec((8, NB, RW), lambda m: (0, m, 0)),
            pl.BlockSpec((1, HID), lambda m: (0, 0)),
        ],
        out_specs=pl.BlockSpec((NB, HID), lambda m: (m, 0)),
        out_shape=jax.ShapeDtypeStruct((N, HID), F32),
    )(big2r, esumr, vcat, asad2, xp2t, b2)


def kernel(x, edge_index, edge_attr, W1, We1, as1, ad1, ae1, b1,
           W2, We2, as2, ad2, ae2, b2):
    src = edge_index[0]
    dst = edge_index[1]

    Ve1 = (We1.reshape(ED, H, C1) * ae1).sum(-1)
    Ve2 = (We2.reshape(ED, H, C2) * ae2).sum(-1)
    vcat = jnp.concatenate([Ve1, Ve2], axis=1)
    vs1 = (W1.reshape(IN, H, C1) * as1).sum(-1)
    vd1 = (W1.reshape(IN, H, C1) * ad1).sum(-1)
    wcat1 = jnp.concatenate(
        [W1, vs1, vd1, jnp.zeros((IN, 8), F32)], axis=1)
    vs2 = (W2.reshape(HID, H, C2) * as2).sum(-1)
    vd2 = (W2.reshape(HID, H, C2) * ad2).sum(-1)
    w2cat = jnp.concatenate([W2, vs2, vd2], axis=1)

    cat1, xp1x, astab1 = _pre(x, wcat1)
    ae12 = _mm(edge_attr, vcat, 8000)
    xp1cat = xp1x.reshape(2 * N, RW)

    wtab1, esum = _sc_wcalc(True, src, dst, ae12, astab1, edge_attr)
    esumr = esum.reshape(NC, NPAD, 16)[:, :N]
    big1f = _sc_layer1(src, dst, wtab1, xp1cat)
    big1r = big1f.reshape(NC, NPAD, RW)[:, :N]

    xp2t, asad2 = _mid(big1r, esumr, cat1, vcat, w2cat,
                       b1.reshape(1, HID))

    wtab2, _unused = _sc_wcalc(False, src, dst, ae12, asad2, edge_attr)
    big2f = _sc_layer2(src, dst, wtab2, xp2t.reshape(8 * N, RW))
    big2r = big2f.reshape(H * NC, NPAD, RW)[:, :N]

    return _post(big2r, esumr, vcat, asad2, xp2t, b2.reshape(1, HID))
